# baseline (device time: 267372 ns/iter reference)
import jax
import jax.numpy as jnp
from jax import lax
from jax.experimental import pallas as pl
from jax.experimental.pallas import tpu as pltpu

N_DEV = 16


def kernel(A, B):
    M, K = A.shape
    _, N = B.shape
    chunk = M // N_DEV

    def body(a_ref, b_ref, out_ref, comm_ref, rs_send, rs_recv, ag_send, ag_recv):
        my = lax.axis_index("i")
        left = lax.rem(my + N_DEV - 1, N_DEV)
        right = lax.rem(my + 1, N_DEV)

        barrier = pltpu.get_barrier_semaphore()
        for nbr in (left, right):
            pl.semaphore_signal(
                barrier, inc=1,
                device_id=(nbr,), device_id_type=pl.DeviceIdType.MESH,
            )
        pl.semaphore_wait(barrier, 2)

        out_ref[:, :] = jnp.dot(
            a_ref[:, :], b_ref[:, :], preferred_element_type=jnp.float32
        )

        for t in range(N_DEV - 1):
            s = lax.rem(my - t + N_DEV, N_DEV)
            r = lax.rem(my - t - 1 + N_DEV, N_DEV)
            rdma = pltpu.make_async_remote_copy(
                src_ref=out_ref.at[pl.ds(s * chunk, chunk), :],
                dst_ref=comm_ref.at[t],
                send_sem=rs_send.at[t],
                recv_sem=rs_recv.at[t],
                device_id=(right,),
                device_id_type=pl.DeviceIdType.MESH,
            )
            rdma.start()
            rdma.wait()
            out_ref[pl.ds(r * chunk, chunk), :] += comm_ref[t]

        mc = lax.rem(my + 1, N_DEV)
        z = out_ref[pl.ds(mc * chunk, chunk), :]
        out_ref[pl.ds(mc * chunk, chunk), :] = z / (1.0 + jnp.exp(-z))

        for t in range(N_DEV - 1):
            cs = lax.rem(my + 1 - t + N_DEV, N_DEV)
            rdma = pltpu.make_async_remote_copy(
                src_ref=out_ref.at[pl.ds(cs * chunk, chunk), :],
                dst_ref=out_ref.at[pl.ds(cs * chunk, chunk), :],
                send_sem=ag_send.at[t],
                recv_sem=ag_recv.at[t],
                device_id=(right,),
                device_id_type=pl.DeviceIdType.MESH,
            )
            rdma.start()
            rdma.wait()

    return pl.pallas_call(
        body,
        out_shape=jax.ShapeDtypeStruct((M, N), jnp.float32),
        in_specs=[
            pl.BlockSpec(memory_space=pltpu.VMEM),
            pl.BlockSpec(memory_space=pltpu.VMEM),
        ],
        out_specs=pl.BlockSpec(memory_space=pltpu.VMEM),
        scratch_shapes=[
            pltpu.VMEM((N_DEV - 1, chunk, N), jnp.float32),
            pltpu.SemaphoreType.DMA((N_DEV - 1,)),
            pltpu.SemaphoreType.DMA((N_DEV - 1,)),
            pltpu.SemaphoreType.DMA((N_DEV - 1,)),
            pltpu.SemaphoreType.DMA((N_DEV - 1,)),
        ],
        compiler_params=pltpu.CompilerParams(
            collective_id=0,
            vmem_limit_bytes=100 * 1024 * 1024,
        ),
    )(A, B)


# device time: 172652 ns/iter; 1.5486x vs baseline; 1.5486x over previous
import jax
import jax.numpy as jnp
from jax import lax
from jax.experimental import pallas as pl
from jax.experimental.pallas import tpu as pltpu

N_DEV = 16


def kernel(A, B):
    M, K = A.shape
    _, N = B.shape
    chunk = M // N_DEV

    def body(a_ref, b_ref, out_ref, z16, comm_ref, rs_send, rs_recv, ag_send, ag_recv):
        my = lax.axis_index("i")
        left = lax.rem(my + N_DEV - 1, N_DEV)
        right = lax.rem(my + 1, N_DEV)

        barrier = pltpu.get_barrier_semaphore()
        for nbr in (left, right):
            pl.semaphore_signal(
                barrier, inc=1,
                device_id=(nbr,), device_id_type=pl.DeviceIdType.MESH,
            )
        pl.semaphore_wait(barrier, 2)

        z16[:, :] = jnp.dot(
            a_ref[:, :], b_ref[:, :], preferred_element_type=jnp.float32
        ).astype(jnp.bfloat16)

        for t in range(N_DEV - 1):
            s = lax.rem(my - t + N_DEV, N_DEV)
            r = lax.rem(my - t - 1 + N_DEV, N_DEV)
            rdma = pltpu.make_async_remote_copy(
                src_ref=z16.at[pl.ds(s * chunk, chunk), :],
                dst_ref=comm_ref.at[t],
                send_sem=rs_send.at[t],
                recv_sem=rs_recv.at[t],
                device_id=(right,),
                device_id_type=pl.DeviceIdType.MESH,
            )
            rdma.start()
            rdma.wait()
            z16[pl.ds(r * chunk, chunk), :] += comm_ref[t]

        mc = lax.rem(my + 1, N_DEV)
        z = z16[pl.ds(mc * chunk, chunk), :].astype(jnp.float32)
        s_out = z / (1.0 + jnp.exp(-z))
        z16[pl.ds(mc * chunk, chunk), :] = s_out.astype(jnp.bfloat16)
        out_ref[pl.ds(mc * chunk, chunk), :] = s_out

        for t in range(N_DEV - 1):
            cs = lax.rem(my + 1 - t + N_DEV, N_DEV)
            rdma = pltpu.make_async_remote_copy(
                src_ref=z16.at[pl.ds(cs * chunk, chunk), :],
                dst_ref=z16.at[pl.ds(cs * chunk, chunk), :],
                send_sem=ag_send.at[t],
                recv_sem=ag_recv.at[t],
                device_id=(right,),
                device_id_type=pl.DeviceIdType.MESH,
            )
            rdma.start()
            rdma.wait()
            cr = lax.rem(my - t + N_DEV, N_DEV)
            out_ref[pl.ds(cr * chunk, chunk), :] = (
                z16[pl.ds(cr * chunk, chunk), :].astype(jnp.float32)
            )

    return pl.pallas_call(
        body,
        out_shape=jax.ShapeDtypeStruct((M, N), jnp.float32),
        in_specs=[
            pl.BlockSpec(memory_space=pltpu.VMEM),
            pl.BlockSpec(memory_space=pltpu.VMEM),
        ],
        out_specs=pl.BlockSpec(memory_space=pltpu.VMEM),
        scratch_shapes=[
            pltpu.VMEM((M, N), jnp.bfloat16),
            pltpu.VMEM((N_DEV - 1, chunk, N), jnp.bfloat16),
            pltpu.SemaphoreType.DMA((N_DEV - 1,)),
            pltpu.SemaphoreType.DMA((N_DEV - 1,)),
            pltpu.SemaphoreType.DMA((N_DEV - 1,)),
            pltpu.SemaphoreType.DMA((N_DEV - 1,)),
        ],
        compiler_params=pltpu.CompilerParams(
            collective_id=0,
            vmem_limit_bytes=100 * 1024 * 1024,
        ),
    )(A, B)


# device time: 164954 ns/iter; 1.6209x vs baseline; 1.0467x over previous
import jax
import jax.numpy as jnp
from jax import lax
from jax.experimental import pallas as pl
from jax.experimental.pallas import tpu as pltpu

N_DEV = 16


def kernel(A, B):
    M, K = A.shape
    _, N = B.shape
    chunk = M // N_DEV
    half = chunk // 2

    def body(a_ref, b_ref, out_ref, z16, comm_r, comm_l,
             r_send, r_recv, l_send, l_recv,
             gr_send, gr_recv, gl_send, gl_recv):
        my = lax.axis_index("i")
        left = lax.rem(my + N_DEV - 1, N_DEV)
        right = lax.rem(my + 1, N_DEV)

        def top(c):
            return pl.ds(c * chunk, half)

        def bot(c):
            return pl.ds(c * chunk + half, half)

        barrier = pltpu.get_barrier_semaphore()
        for nbr in (left, right):
            pl.semaphore_signal(
                barrier, inc=1,
                device_id=(nbr,), device_id_type=pl.DeviceIdType.MESH,
            )
        pl.semaphore_wait(barrier, 2)

        z16[:, :] = jnp.dot(
            a_ref[:, :], b_ref[:, :], preferred_element_type=jnp.float32
        ).astype(jnp.bfloat16)

        for t in range(N_DEV - 1):
            sr = lax.rem(my - t + N_DEV, N_DEV)
            rr = lax.rem(my - t - 1 + N_DEV, N_DEV)
            sl = lax.rem(my + t, N_DEV)
            rl = lax.rem(my + t + 1, N_DEV)
            rdma_r = pltpu.make_async_remote_copy(
                src_ref=z16.at[top(sr), :],
                dst_ref=comm_r.at[t],
                send_sem=r_send.at[t],
                recv_sem=r_recv.at[t],
                device_id=(right,),
                device_id_type=pl.DeviceIdType.MESH,
            )
            rdma_l = pltpu.make_async_remote_copy(
                src_ref=z16.at[bot(sl), :],
                dst_ref=comm_l.at[t],
                send_sem=l_send.at[t],
                recv_sem=l_recv.at[t],
                device_id=(left,),
                device_id_type=pl.DeviceIdType.MESH,
            )
            rdma_r.start()
            rdma_l.start()
            rdma_r.wait()
            z16[top(rr), :] += comm_r[t]
            rdma_l.wait()
            z16[bot(rl), :] += comm_l[t]

        mc_r = lax.rem(my + 1, N_DEV)
        mc_l = lax.rem(my + N_DEV - 1, N_DEV)
        zr = z16[top(mc_r), :].astype(jnp.float32)
        sr_out = zr / (1.0 + jnp.exp(-zr))
        z16[top(mc_r), :] = sr_out.astype(jnp.bfloat16)
        out_ref[top(mc_r), :] = sr_out
        zl = z16[bot(mc_l), :].astype(jnp.float32)
        sl_out = zl / (1.0 + jnp.exp(-zl))
        z16[bot(mc_l), :] = sl_out.astype(jnp.bfloat16)
        out_ref[bot(mc_l), :] = sl_out

        def ag_start(t):
            cr = lax.rem(my + 1 - t + N_DEV, N_DEV)
            cl = lax.rem(my - 1 + t + N_DEV, N_DEV)
            rdma_r = pltpu.make_async_remote_copy(
                src_ref=z16.at[top(cr), :],
                dst_ref=z16.at[top(cr), :],
                send_sem=gr_send.at[t],
                recv_sem=gr_recv.at[t],
                device_id=(right,),
                device_id_type=pl.DeviceIdType.MESH,
            )
            rdma_l = pltpu.make_async_remote_copy(
                src_ref=z16.at[bot(cl), :],
                dst_ref=z16.at[bot(cl), :],
                send_sem=gl_send.at[t],
                recv_sem=gl_recv.at[t],
                device_id=(left,),
                device_id_type=pl.DeviceIdType.MESH,
            )
            rdma_r.start()
            rdma_l.start()
            return rdma_r, rdma_l

        pending = ag_start(0)
        for t in range(N_DEV - 1):
            rdma_r, rdma_l = pending
            rdma_r.wait()
            rdma_l.wait()
            if t + 1 < N_DEV - 1:
                pending = ag_start(t + 1)
            cr = lax.rem(my - t + N_DEV, N_DEV)
            cl = lax.rem(my + t, N_DEV)
            out_ref[top(cr), :] = z16[top(cr), :].astype(jnp.float32)
            out_ref[bot(cl), :] = z16[bot(cl), :].astype(jnp.float32)

    return pl.pallas_call(
        body,
        out_shape=jax.ShapeDtypeStruct((M, N), jnp.float32),
        in_specs=[
            pl.BlockSpec(memory_space=pltpu.VMEM),
            pl.BlockSpec(memory_space=pltpu.VMEM),
        ],
        out_specs=pl.BlockSpec(memory_space=pltpu.VMEM),
        scratch_shapes=[
            pltpu.VMEM((M, N), jnp.bfloat16),
            pltpu.VMEM((N_DEV - 1, half, N), jnp.bfloat16),
            pltpu.VMEM((N_DEV - 1, half, N), jnp.bfloat16),
            pltpu.SemaphoreType.DMA((N_DEV - 1,)),
            pltpu.SemaphoreType.DMA((N_DEV - 1,)),
            pltpu.SemaphoreType.DMA((N_DEV - 1,)),
            pltpu.SemaphoreType.DMA((N_DEV - 1,)),
            pltpu.SemaphoreType.DMA((N_DEV - 1,)),
            pltpu.SemaphoreType.DMA((N_DEV - 1,)),
            pltpu.SemaphoreType.DMA((N_DEV - 1,)),
            pltpu.SemaphoreType.DMA((N_DEV - 1,)),
        ],
        compiler_params=pltpu.CompilerParams(
            collective_id=0,
            vmem_limit_bytes=100 * 1024 * 1024,
        ),
    )(A, B)


# device time: 101723 ns/iter; 2.6284x vs baseline; 1.6216x over previous
import jax
import jax.numpy as jnp
from jax import lax
from jax.experimental import pallas as pl
from jax.experimental.pallas import tpu as pltpu

N_DEV = 16
PLANE = 4
NZ = 4


def kernel(A, B):
    M, K = A.shape
    _, N = B.shape
    pch = M // PLANE
    half = pch // 2
    q = half // 2
    e = q // 2

    def body(a_ref, b_ref, out_ref, z16, comm_r, comm_l, comm_z1, comm_z2,
             ar_send, ar_recv, al_send, al_recv,
             z1_send, z1_recv, z2_send, z2_recv,
             g2_send, g2_recv, g1_send, g1_recv,
             gr_send, gr_recv, gl_send, gl_recv):
        my = lax.axis_index("i")
        w = lax.rem(my, PLANE)
        zz = lax.div(my, PLANE)
        zb = zz * PLANE
        p_right = zb + lax.rem(w + 1, PLANE)
        p_left = zb + lax.rem(w + PLANE - 1, PLANE)
        bz1 = lax.rem(zz, 2)
        bz2 = lax.div(zz, 2)
        z1p = my + 4 - 8 * bz1
        z2p = my + 8 - 16 * bz2

        barrier = pltpu.get_barrier_semaphore()
        for nbr in (p_left, p_right, z1p, z2p):
            pl.semaphore_signal(
                barrier, inc=1,
                device_id=(nbr,), device_id_type=pl.DeviceIdType.MESH,
            )
        pl.semaphore_wait(barrier, 4)

        z16[:, :] = jnp.dot(
            a_ref[:, :], b_ref[:, :], preferred_element_type=jnp.float32
        ).astype(jnp.bfloat16)

        def exch(src_rows, dst, dst_rows_or_slot, ssem, rsem, peer, direct):
            if direct:
                dst_ref = z16.at[pl.ds(dst_rows_or_slot, src_rows[1]), :]
            else:
                dst_ref = dst.at[dst_rows_or_slot]
            return pltpu.make_async_remote_copy(
                src_ref=z16.at[pl.ds(src_rows[0], src_rows[1]), :],
                dst_ref=dst_ref,
                send_sem=ssem,
                recv_sem=rsem,
                device_id=(peer,),
                device_id_type=pl.DeviceIdType.MESH,
            )

        for t in range(PLANE - 1):
            sr = lax.rem(w - t + PLANE, PLANE)
            rr = lax.rem(w - t - 1 + PLANE, PLANE)
            sl = lax.rem(w + t, PLANE)
            rl = lax.rem(w + t + 1, PLANE)
            rd_r = exch((sr * pch, half), comm_r, t,
                        ar_send.at[t], ar_recv.at[t], p_right, False)
            rd_l = exch((sl * pch + half, half), comm_l, t,
                        al_send.at[t], al_recv.at[t], p_left, False)
            rd_r.start()
            rd_l.start()
            rd_r.wait()
            z16[pl.ds(rr * pch, half), :] += comm_r[t]
            rd_l.wait()
            z16[pl.ds(rl * pch + half, half), :] += comm_l[t]

        tc = lax.rem(w + 1, PLANE)
        bc = lax.rem(w + PLANE - 1, PLANE)
        tb = tc * pch
        bb = bc * pch + half

        rd_t = exch((tb + (1 - bz1) * q, q), comm_z1, 0,
                    z1_send.at[0], z1_recv.at[0], z1p, False)
        rd_b = exch((bb + (1 - bz1) * q, q), comm_z1, 1,
                    z1_send.at[1], z1_recv.at[1], z1p, False)
        rd_t.start()
        rd_b.start()
        rd_t.wait()
        z16[pl.ds(tb + bz1 * q, q), :] += comm_z1[0]
        rd_b.wait()
        z16[pl.ds(bb + bz1 * q, q), :] += comm_z1[1]

        tb2 = tb + bz1 * q
        bb2 = bb + bz1 * q
        rd_t = exch((tb2 + (1 - bz2) * e, e), comm_z2, 0,
                    z2_send.at[0], z2_recv.at[0], z2p, False)
        rd_b = exch((bb2 + (1 - bz2) * e, e), comm_z2, 1,
                    z2_send.at[1], z2_recv.at[1], z2p, False)
        rd_t.start()
        rd_b.start()
        rd_t.wait()
        z16[pl.ds(tb2 + bz2 * e, e), :] += comm_z2[0]
        rd_b.wait()
        z16[pl.ds(bb2 + bz2 * e, e), :] += comm_z2[1]

        tf = tb2 + bz2 * e
        bf = bb2 + bz2 * e
        for base in (tf, bf):
            zv = z16[pl.ds(base, e), :].astype(jnp.float32)
            sv = zv / (1.0 + jnp.exp(-zv))
            z16[pl.ds(base, e), :] = sv.astype(jnp.bfloat16)
            out_ref[pl.ds(base, e), :] = sv

        rd_t = exch((tf, e), None, tf, g2_send.at[0], g2_recv.at[0], z2p, True)
        rd_b = exch((bf, e), None, bf, g2_send.at[1], g2_recv.at[1], z2p, True)
        rd_t.start()
        rd_b.start()
        rd_t.wait()
        rd_b.wait()
        rd_t = exch((tb2, q), None, tb2, g1_send.at[0], g1_recv.at[0], z1p, True)
        rd_b = exch((bb2, q), None, bb2, g1_send.at[1], g1_recv.at[1], z1p, True)
        rd_t.start()
        rd_b.start()
        out_ref[pl.ds(tb2 + (1 - bz2) * e, e), :] = (
            z16[pl.ds(tb2 + (1 - bz2) * e, e), :].astype(jnp.float32))
        out_ref[pl.ds(bb2 + (1 - bz2) * e, e), :] = (
            z16[pl.ds(bb2 + (1 - bz2) * e, e), :].astype(jnp.float32))
        rd_t.wait()
        rd_b.wait()

        def ag_start(t):
            cr = lax.rem(w + 1 - t + PLANE, PLANE)
            cl = lax.rem(w - 1 + t + PLANE, PLANE)
            rd_r = exch((cr * pch, half), None, cr * pch,
                        gr_send.at[t], gr_recv.at[t], p_right, True)
            rd_l = exch((cl * pch + half, half), None, cl * pch + half,
                        gl_send.at[t], gl_recv.at[t], p_left, True)
            rd_r.start()
            rd_l.start()
            return rd_r, rd_l

        pending = ag_start(0)
        out_ref[pl.ds(tb + (1 - bz1) * q, q), :] = (
            z16[pl.ds(tb + (1 - bz1) * q, q), :].astype(jnp.float32))
        out_ref[pl.ds(bb + (1 - bz1) * q, q), :] = (
            z16[pl.ds(bb + (1 - bz1) * q, q), :].astype(jnp.float32))
        for t in range(PLANE - 1):
            rd_r, rd_l = pending
            rd_r.wait()
            rd_l.wait()
            if t + 1 < PLANE - 1:
                pending = ag_start(t + 1)
            cr = lax.rem(w - t + PLANE, PLANE)
            cl = lax.rem(w + t, PLANE)
            out_ref[pl.ds(cr * pch, half), :] = (
                z16[pl.ds(cr * pch, half), :].astype(jnp.float32))
            out_ref[pl.ds(cl * pch + half, half), :] = (
                z16[pl.ds(cl * pch + half, half), :].astype(jnp.float32))

    n_steps = PLANE - 1
    return pl.pallas_call(
        body,
        out_shape=jax.ShapeDtypeStruct((M, N), jnp.float32),
        in_specs=[
            pl.BlockSpec(memory_space=pltpu.VMEM),
            pl.BlockSpec(memory_space=pltpu.VMEM),
        ],
        out_specs=pl.BlockSpec(memory_space=pltpu.VMEM),
        scratch_shapes=[
            pltpu.VMEM((M, N), jnp.bfloat16),
            pltpu.VMEM((n_steps, half, N), jnp.bfloat16),
            pltpu.VMEM((n_steps, half, N), jnp.bfloat16),
            pltpu.VMEM((2, q, N), jnp.bfloat16),
            pltpu.VMEM((2, e, N), jnp.bfloat16),
            pltpu.SemaphoreType.DMA((n_steps,)),
            pltpu.SemaphoreType.DMA((n_steps,)),
            pltpu.SemaphoreType.DMA((n_steps,)),
            pltpu.SemaphoreType.DMA((n_steps,)),
            pltpu.SemaphoreType.DMA((2,)),
            pltpu.SemaphoreType.DMA((2,)),
            pltpu.SemaphoreType.DMA((2,)),
            pltpu.SemaphoreType.DMA((2,)),
            pltpu.SemaphoreType.DMA((2,)),
            pltpu.SemaphoreType.DMA((2,)),
            pltpu.SemaphoreType.DMA((2,)),
            pltpu.SemaphoreType.DMA((2,)),
            pltpu.SemaphoreType.DMA((n_steps,)),
            pltpu.SemaphoreType.DMA((n_steps,)),
            pltpu.SemaphoreType.DMA((n_steps,)),
            pltpu.SemaphoreType.DMA((n_steps,)),
        ],
        compiler_params=pltpu.CompilerParams(
            collective_id=0,
            vmem_limit_bytes=100 * 1024 * 1024,
        ),
    )(A, B)


# device time: 98809 ns/iter; 2.7059x vs baseline; 1.0295x over previous
import jax
import jax.numpy as jnp
from jax import lax
from jax.experimental import pallas as pl
from jax.experimental.pallas import tpu as pltpu

N_DEV = 16
PLANE = 4
NZ = 4


def kernel(A, B):
    M, K = A.shape
    _, N = B.shape
    pch = M // PLANE
    half = pch // 2
    q = half // 2
    e = q // 2

    def body(a_ref, b_ref, out_ref, z16, comm_r, comm_l, comm_z1, comm_z2,
             ar_send, ar_recv, al_send, al_recv,
             z1_send, z1_recv, z2_send, z2_recv,
             g2_send, g2_recv, g1_send, g1_recv,
             gr_send, gr_recv, gl_send, gl_recv):
        my = lax.axis_index("i")
        w = lax.rem(my, PLANE)
        zz = lax.div(my, PLANE)
        zb = zz * PLANE
        p_right = zb + lax.rem(w + 1, PLANE)
        p_left = zb + lax.rem(w + PLANE - 1, PLANE)
        bz1 = lax.rem(zz, 2)
        bz2 = lax.div(zz, 2)
        z1p = my + 4 - 8 * bz1
        z2p = my + 8 - 16 * bz2

        barrier = pltpu.get_barrier_semaphore()
        for nbr in (p_left, p_right, z1p, z2p):
            pl.semaphore_signal(
                barrier, inc=1,
                device_id=(nbr,), device_id_type=pl.DeviceIdType.MESH,
            )
        pl.semaphore_wait(barrier, 4)

        def mm_slab(c):
            z16[pl.ds(c * pch, pch), :] = jnp.dot(
                a_ref[pl.ds(c * pch, pch), :], b_ref[:, :],
                preferred_element_type=jnp.float32,
            ).astype(jnp.bfloat16)

        def exch(src_rows, dst, dst_rows_or_slot, ssem, rsem, peer, direct):
            if direct:
                dst_ref = z16.at[pl.ds(dst_rows_or_slot, src_rows[1]), :]
            else:
                dst_ref = dst.at[dst_rows_or_slot]
            return pltpu.make_async_remote_copy(
                src_ref=z16.at[pl.ds(src_rows[0], src_rows[1]), :],
                dst_ref=dst_ref,
                send_sem=ssem,
                recv_sem=rsem,
                device_id=(peer,),
                device_id_type=pl.DeviceIdType.MESH,
            )

        mm_slab(w)
        for t in range(PLANE - 1):
            sr = lax.rem(w - t + PLANE, PLANE)
            rr = lax.rem(w - t - 1 + PLANE, PLANE)
            sl = lax.rem(w + t, PLANE)
            rl = lax.rem(w + t + 1, PLANE)
            rd_r = exch((sr * pch, half), comm_r, t,
                        ar_send.at[t], ar_recv.at[t], p_right, False)
            rd_l = exch((sl * pch + half, half), comm_l, t,
                        al_send.at[t], al_recv.at[t], p_left, False)
            rd_r.start()
            rd_l.start()
            if t == 0:
                mm_slab(lax.rem(w + PLANE - 1, PLANE))
                mm_slab(lax.rem(w + 1, PLANE))
                mm_slab(lax.rem(w + 2, PLANE))
            rd_r.wait()
            z16[pl.ds(rr * pch, half), :] += comm_r[t]
            rd_l.wait()
            z16[pl.ds(rl * pch + half, half), :] += comm_l[t]

        tc = lax.rem(w + 1, PLANE)
        bc = lax.rem(w + PLANE - 1, PLANE)
        tb = tc * pch
        bb = bc * pch + half

        rd_t = exch((tb + (1 - bz1) * q, q), comm_z1, 0,
                    z1_send.at[0], z1_recv.at[0], z1p, False)
        rd_b = exch((bb + (1 - bz1) * q, q), comm_z1, 1,
                    z1_send.at[1], z1_recv.at[1], z1p, False)
        rd_t.start()
        rd_b.start()
        rd_t.wait()
        z16[pl.ds(tb + bz1 * q, q), :] += comm_z1[0]
        rd_b.wait()
        z16[pl.ds(bb + bz1 * q, q), :] += comm_z1[1]

        tb2 = tb + bz1 * q
        bb2 = bb + bz1 * q
        rd_t = exch((tb2 + (1 - bz2) * e, e), comm_z2, 0,
                    z2_send.at[0], z2_recv.at[0], z2p, False)
        rd_b = exch((bb2 + (1 - bz2) * e, e), comm_z2, 1,
                    z2_send.at[1], z2_recv.at[1], z2p, False)
        rd_t.start()
        rd_b.start()
        rd_t.wait()
        z16[pl.ds(tb2 + bz2 * e, e), :] += comm_z2[0]
        rd_b.wait()
        z16[pl.ds(bb2 + bz2 * e, e), :] += comm_z2[1]

        tf = tb2 + bz2 * e
        bf = bb2 + bz2 * e
        for base in (tf, bf):
            zv = z16[pl.ds(base, e), :].astype(jnp.float32)
            sv = zv / (1.0 + jnp.exp(-zv))
            z16[pl.ds(base, e), :] = sv.astype(jnp.bfloat16)
            out_ref[pl.ds(base, e), :] = sv

        rd_t = exch((tf, e), None, tf, g2_send.at[0], g2_recv.at[0], z2p, True)
        rd_b = exch((bf, e), None, bf, g2_send.at[1], g2_recv.at[1], z2p, True)
        rd_t.start()
        rd_b.start()
        rd_t.wait()
        rd_b.wait()
        rd_t = exch((tb2, q), None, tb2, g1_send.at[0], g1_recv.at[0], z1p, True)
        rd_b = exch((bb2, q), None, bb2, g1_send.at[1], g1_recv.at[1], z1p, True)
        rd_t.start()
        rd_b.start()
        out_ref[pl.ds(tb2 + (1 - bz2) * e, e), :] = (
            z16[pl.ds(tb2 + (1 - bz2) * e, e), :].astype(jnp.float32))
        out_ref[pl.ds(bb2 + (1 - bz2) * e, e), :] = (
            z16[pl.ds(bb2 + (1 - bz2) * e, e), :].astype(jnp.float32))
        rd_t.wait()
        rd_b.wait()

        def ag_start(t):
            cr = lax.rem(w + 1 - t + PLANE, PLANE)
            cl = lax.rem(w - 1 + t + PLANE, PLANE)
            rd_r = exch((cr * pch, half), None, cr * pch,
                        gr_send.at[t], gr_recv.at[t], p_right, True)
            rd_l = exch((cl * pch + half, half), None, cl * pch + half,
                        gl_send.at[t], gl_recv.at[t], p_left, True)
            rd_r.start()
            rd_l.start()
            return rd_r, rd_l

        pending = ag_start(0)
        out_ref[pl.ds(tb + (1 - bz1) * q, q), :] = (
            z16[pl.ds(tb + (1 - bz1) * q, q), :].astype(jnp.float32))
        out_ref[pl.ds(bb + (1 - bz1) * q, q), :] = (
            z16[pl.ds(bb + (1 - bz1) * q, q), :].astype(jnp.float32))
        for t in range(PLANE - 1):
            rd_r, rd_l = pending
            rd_r.wait()
            rd_l.wait()
            if t + 1 < PLANE - 1:
                pending = ag_start(t + 1)
            cr = lax.rem(w - t + PLANE, PLANE)
            cl = lax.rem(w + t, PLANE)
            out_ref[pl.ds(cr * pch, half), :] = (
                z16[pl.ds(cr * pch, half), :].astype(jnp.float32))
            out_ref[pl.ds(cl * pch + half, half), :] = (
                z16[pl.ds(cl * pch + half, half), :].astype(jnp.float32))

    n_steps = PLANE - 1
    return pl.pallas_call(
        body,
        out_shape=jax.ShapeDtypeStruct((M, N), jnp.float32),
        in_specs=[
            pl.BlockSpec(memory_space=pltpu.VMEM),
            pl.BlockSpec(memory_space=pltpu.VMEM),
        ],
        out_specs=pl.BlockSpec(memory_space=pltpu.VMEM),
        scratch_shapes=[
            pltpu.VMEM((M, N), jnp.bfloat16),
            pltpu.VMEM((n_steps, half, N), jnp.bfloat16),
            pltpu.VMEM((n_steps, half, N), jnp.bfloat16),
            pltpu.VMEM((2, q, N), jnp.bfloat16),
            pltpu.VMEM((2, e, N), jnp.bfloat16),
            pltpu.SemaphoreType.DMA((n_steps,)),
            pltpu.SemaphoreType.DMA((n_steps,)),
            pltpu.SemaphoreType.DMA((n_steps,)),
            pltpu.SemaphoreType.DMA((n_steps,)),
            pltpu.SemaphoreType.DMA((2,)),
            pltpu.SemaphoreType.DMA((2,)),
            pltpu.SemaphoreType.DMA((2,)),
            pltpu.SemaphoreType.DMA((2,)),
            pltpu.SemaphoreType.DMA((2,)),
            pltpu.SemaphoreType.DMA((2,)),
            pltpu.SemaphoreType.DMA((2,)),
            pltpu.SemaphoreType.DMA((2,)),
            pltpu.SemaphoreType.DMA((n_steps,)),
            pltpu.SemaphoreType.DMA((n_steps,)),
            pltpu.SemaphoreType.DMA((n_steps,)),
            pltpu.SemaphoreType.DMA((n_steps,)),
        ],
        compiler_params=pltpu.CompilerParams(
            collective_id=0,
            vmem_limit_bytes=100 * 1024 * 1024,
        ),
    )(A, B)


# device time: 89602 ns/iter; 2.9840x vs baseline; 1.1028x over previous
import jax
import jax.numpy as jnp
from jax import lax
from jax.experimental import pallas as pl
from jax.experimental.pallas import tpu as pltpu

N_DEV = 16
PLANE = 4
NZ = 4


def kernel(A, B):
    M, K = A.shape
    _, N = B.shape
    pch = M // PLANE
    half = pch // 2
    q = half // 2
    e = q // 2

    def body(a_ref, b_ref, out_ref, z16, comm_r, comm_l, comm_z1, comm_z2,
             ar_send, ar_recv, al_send, al_recv,
             z1_send, z1_recv, z2_send, z2_recv,
             g2_send, g2_recv, g1_send, g1_recv,
             ags, agr):
        my = lax.axis_index("i")
        w = lax.rem(my, PLANE)
        zz = lax.div(my, PLANE)
        zb = zz * PLANE
        p_right = zb + lax.rem(w + 1, PLANE)
        p_left = zb + lax.rem(w + PLANE - 1, PLANE)
        bz1 = lax.rem(zz, 2)
        bz2 = lax.div(zz, 2)
        z1p = my + 4 - 8 * bz1
        z2p = my + 8 - 16 * bz2

        barrier = pltpu.get_barrier_semaphore()
        for nbr in (p_left, p_right, z1p, z2p):
            pl.semaphore_signal(
                barrier, inc=1,
                device_id=(nbr,), device_id_type=pl.DeviceIdType.MESH,
            )
        pl.semaphore_wait(barrier, 4)

        def mm_slab(c):
            z16[pl.ds(c * pch, pch), :] = jnp.dot(
                a_ref[pl.ds(c * pch, pch), :], b_ref[:, :],
                preferred_element_type=jnp.float32,
            ).astype(jnp.bfloat16)

        def exch(src_rows, dst, dst_rows_or_slot, ssem, rsem, peer, direct):
            if direct:
                dst_ref = z16.at[pl.ds(dst_rows_or_slot, src_rows[1]), :]
            else:
                dst_ref = dst.at[dst_rows_or_slot]
            return pltpu.make_async_remote_copy(
                src_ref=z16.at[pl.ds(src_rows[0], src_rows[1]), :],
                dst_ref=dst_ref,
                send_sem=ssem,
                recv_sem=rsem,
                device_id=(peer,),
                device_id_type=pl.DeviceIdType.MESH,
            )

        mm_slab(w)
        for t in range(PLANE - 1):
            sr = lax.rem(w - t + PLANE, PLANE)
            rr = lax.rem(w - t - 1 + PLANE, PLANE)
            sl = lax.rem(w + t, PLANE)
            rl = lax.rem(w + t + 1, PLANE)
            rd_r = exch((sr * pch, half), comm_r, t,
                        ar_send.at[t], ar_recv.at[t], p_right, False)
            rd_l = exch((sl * pch + half, half), comm_l, t,
                        al_send.at[t], al_recv.at[t], p_left, False)
            rd_r.start()
            rd_l.start()
            if t == 0:
                mm_slab(lax.rem(w + PLANE - 1, PLANE))
                mm_slab(lax.rem(w + 1, PLANE))
                mm_slab(lax.rem(w + 2, PLANE))
            rd_r.wait()
            z16[pl.ds(rr * pch, half), :] += comm_r[t]
            rd_l.wait()
            z16[pl.ds(rl * pch + half, half), :] += comm_l[t]

        tc = lax.rem(w + 1, PLANE)
        bc = lax.rem(w + PLANE - 1, PLANE)
        tb = tc * pch
        bb = bc * pch + half

        rd_t = exch((tb + (1 - bz1) * q, q), comm_z1, 0,
                    z1_send.at[0], z1_recv.at[0], z1p, False)
        rd_b = exch((bb + (1 - bz1) * q, q), comm_z1, 1,
                    z1_send.at[1], z1_recv.at[1], z1p, False)
        rd_t.start()
        rd_b.start()
        rd_t.wait()
        z16[pl.ds(tb + bz1 * q, q), :] += comm_z1[0]
        rd_b.wait()
        z16[pl.ds(bb + bz1 * q, q), :] += comm_z1[1]

        tb2 = tb + bz1 * q
        bb2 = bb + bz1 * q
        rd_t = exch((tb2 + (1 - bz2) * e, e), comm_z2, 0,
                    z2_send.at[0], z2_recv.at[0], z2p, False)
        rd_b = exch((bb2 + (1 - bz2) * e, e), comm_z2, 1,
                    z2_send.at[1], z2_recv.at[1], z2p, False)
        rd_t.start()
        rd_b.start()
        rd_t.wait()
        z16[pl.ds(tb2 + bz2 * e, e), :] += comm_z2[0]
        rd_b.wait()
        z16[pl.ds(bb2 + bz2 * e, e), :] += comm_z2[1]

        tf = tb2 + bz2 * e
        bf = bb2 + bz2 * e
        for base in (tf, bf):
            zv = z16[pl.ds(base, e), :].astype(jnp.float32)
            sv = zv / (1.0 + jnp.exp(-zv))
            z16[pl.ds(base, e), :] = sv.astype(jnp.bfloat16)
            out_ref[pl.ds(base, e), :] = sv

        off0 = bz1 * q + bz2 * e
        off1 = bz1 * q + (1 - bz2) * e
        offL = (1 - bz1) * q

        def sub_info(kind, t):
            off, sz = ((off0, e), (off1, e), (offL, q))[kind % 3]
            if kind < 3:
                cs = lax.rem(w + 1 - t + PLANE, PLANE)
                cr_ = lax.rem(w - t + PLANE, PLANE)
                extra, peer = 0, p_right
            else:
                cs = lax.rem(w - 1 + t + PLANE, PLANE)
                cr_ = lax.rem(w + t, PLANE)
                extra, peer = half, p_left
            return cs * pch + extra + off, cr_ * pch + extra + off, sz, peer

        def sub_start(kind, t):
            sbase, _, sz, peer = sub_info(kind, t)
            rd = exch((sbase, sz), None, sbase,
                      ags.at[kind, t], agr.at[kind, t], peer, True)
            rd.start()
            return rd

        def conv(base, sz):
            out_ref[pl.ds(base, sz), :] = (
                z16[pl.ds(base, sz), :].astype(jnp.float32))

        def sub_finish(kind, t, rd):
            rd.wait()
            nxt = sub_start(kind, t + 1) if t + 1 < PLANE - 1 else None
            _, rbase, sz, _ = sub_info(kind, t)
            conv(rbase, sz)
            return nxt

        r0 = sub_start(0, 0)
        l0 = sub_start(3, 0)
        rd_t = exch((tf, e), None, tf, g2_send.at[0], g2_recv.at[0], z2p, True)
        rd_b = exch((bf, e), None, bf, g2_send.at[1], g2_recv.at[1], z2p, True)
        rd_t.start()
        rd_b.start()
        rd_t.wait()
        rd_b.wait()
        zd_t = exch((tb2, q), None, tb2, g1_send.at[0], g1_recv.at[0], z1p, True)
        zd_b = exch((bb2, q), None, bb2, g1_send.at[1], g1_recv.at[1], z1p, True)
        zd_t.start()
        zd_b.start()
        r1 = sub_start(1, 0)
        l1 = sub_start(4, 0)
        conv(tb2 + (1 - bz2) * e, e)
        conv(bb2 + (1 - bz2) * e, e)
        r0 = sub_finish(0, 0, r0)
        l0 = sub_finish(3, 0, l0)
        r1 = sub_finish(1, 0, r1)
        l1 = sub_finish(4, 0, l1)
        r0 = sub_finish(0, 1, r0)
        l0 = sub_finish(3, 1, l0)
        zd_t.wait()
        zd_b.wait()
        rL = sub_start(2, 0)
        lL = sub_start(5, 0)
        conv(tb + (1 - bz1) * q, q)
        conv(bb + (1 - bz1) * q, q)
        r1 = sub_finish(1, 1, r1)
        l1 = sub_finish(4, 1, l1)
        r0 = sub_finish(0, 2, r0)
        l0 = sub_finish(3, 2, l0)
        rL = sub_finish(2, 0, rL)
        lL = sub_finish(5, 0, lL)
        r1 = sub_finish(1, 2, r1)
        l1 = sub_finish(4, 2, l1)
        rL = sub_finish(2, 1, rL)
        lL = sub_finish(5, 1, lL)
        rL = sub_finish(2, 2, rL)
        lL = sub_finish(5, 2, lL)

    n_steps = PLANE - 1
    return pl.pallas_call(
        body,
        out_shape=jax.ShapeDtypeStruct((M, N), jnp.float32),
        in_specs=[
            pl.BlockSpec(memory_space=pltpu.VMEM),
            pl.BlockSpec(memory_space=pltpu.VMEM),
        ],
        out_specs=pl.BlockSpec(memory_space=pltpu.VMEM),
        scratch_shapes=[
            pltpu.VMEM((M, N), jnp.bfloat16),
            pltpu.VMEM((n_steps, half, N), jnp.bfloat16),
            pltpu.VMEM((n_steps, half, N), jnp.bfloat16),
            pltpu.VMEM((2, q, N), jnp.bfloat16),
            pltpu.VMEM((2, e, N), jnp.bfloat16),
            pltpu.SemaphoreType.DMA((n_steps,)),
            pltpu.SemaphoreType.DMA((n_steps,)),
            pltpu.SemaphoreType.DMA((n_steps,)),
            pltpu.SemaphoreType.DMA((n_steps,)),
            pltpu.SemaphoreType.DMA((2,)),
            pltpu.SemaphoreType.DMA((2,)),
            pltpu.SemaphoreType.DMA((2,)),
            pltpu.SemaphoreType.DMA((2,)),
            pltpu.SemaphoreType.DMA((2,)),
            pltpu.SemaphoreType.DMA((2,)),
            pltpu.SemaphoreType.DMA((2,)),
            pltpu.SemaphoreType.DMA((2,)),
            pltpu.SemaphoreType.DMA((6, n_steps)),
            pltpu.SemaphoreType.DMA((6, n_steps)),
        ],
        compiler_params=pltpu.CompilerParams(
            collective_id=0,
            vmem_limit_bytes=100 * 1024 * 1024,
        ),
    )(A, B)


# device time: 82921 ns/iter; 3.2244x vs baseline; 1.0806x over previous
import jax
import jax.numpy as jnp
from jax import lax
from jax.experimental import pallas as pl
from jax.experimental.pallas import tpu as pltpu

N_DEV = 16
PLANE = 4


def kernel(A, B):
    M, K = A.shape
    _, N = B.shape
    pch = M // PLANE
    half = pch // 2
    q = half // 2
    e = q // 2

    def body(a_ref, b_ref, out_ref, z16, rs_comm, comm_z1, comm_z2,
             ars, arr, z1_send, z1_recv, z2_send, z2_recv,
             g2_send, g2_recv, g1_send, g1_recv, ags, agr):
        my = lax.axis_index("i")
        w = lax.rem(my, PLANE)
        zz = lax.div(my, PLANE)
        zb = zz * PLANE
        p_right = zb + lax.rem(w + 1, PLANE)
        p_left = zb + lax.rem(w + PLANE - 1, PLANE)
        bz1 = lax.rem(zz, 2)
        bz2 = lax.div(zz, 2)
        z1p = my + 4 - 8 * bz1
        z2p = my + 8 - 16 * bz2

        barrier = pltpu.get_barrier_semaphore()
        for nbr in (p_left, p_right, z1p, z2p):
            pl.semaphore_signal(
                barrier, inc=1,
                device_id=(nbr,), device_id_type=pl.DeviceIdType.MESH,
            )
        pl.semaphore_wait(barrier, 4)

        def exch(src_rows, dst_ref, ssem, rsem, peer):
            return pltpu.make_async_remote_copy(
                src_ref=z16.at[pl.ds(src_rows[0], src_rows[1]), :],
                dst_ref=dst_ref,
                send_sem=ssem,
                recv_sem=rsem,
                device_id=(peer,),
                device_id_type=pl.DeviceIdType.MESH,
            )

        def mm_slab(c):
            z16[pl.ds(c * pch, pch), :] = jnp.dot(
                a_ref[pl.ds(c * pch, pch), :], b_ref[:, :],
                preferred_element_type=jnp.float32,
            ).astype(jnp.bfloat16)

        offa = (1 - bz1) * q
        offb = bz1 * q

        def rs_info(kind, t):
            off = offa if kind % 2 == 0 else offb
            if kind < 2:
                cs = lax.rem(w - t + PLANE, PLANE)
                cr_ = lax.rem(w - t - 1 + PLANE, PLANE)
                extra, peer = 0, p_right
            else:
                cs = lax.rem(w + t, PLANE)
                cr_ = lax.rem(w + t + 1, PLANE)
                extra, peer = half, p_left
            return cs * pch + extra + off, cr_ * pch + extra + off, peer

        def rs_start(kind, t):
            sbase, _, peer = rs_info(kind, t)
            rd = exch((sbase, q), rs_comm.at[kind, t],
                      ars.at[kind, t], arr.at[kind, t], peer)
            rd.start()
            return rd

        def rs_finish(kind, t, rd):
            rd.wait()
            _, rbase, _ = rs_info(kind, t)
            z16[pl.ds(rbase, q), :] += rs_comm[kind, t]
            return rs_start(kind, t + 1) if t + 1 < PLANE - 1 else None

        mm_slab(w)
        ra = rs_start(0, 0)
        la = rs_start(2, 0)
        rb = rs_start(1, 0)
        lb = rs_start(3, 0)
        mm_slab(lax.rem(w + PLANE - 1, PLANE))
        mm_slab(lax.rem(w + 1, PLANE))
        mm_slab(lax.rem(w + 2, PLANE))
        ra = rs_finish(0, 0, ra)
        la = rs_finish(2, 0, la)
        rb = rs_finish(1, 0, rb)
        lb = rs_finish(3, 0, lb)
        ra = rs_finish(0, 1, ra)
        la = rs_finish(2, 1, la)
        rb = rs_finish(1, 1, rb)
        lb = rs_finish(3, 1, lb)
        rs_finish(0, 2, ra)
        rs_finish(2, 2, la)

        tc = lax.rem(w + 1, PLANE)
        bc = lax.rem(w + PLANE - 1, PLANE)
        tb = tc * pch
        bb = bc * pch + half

        zt = exch((tb + offa, q), comm_z1.at[0],
                  z1_send.at[0], z1_recv.at[0], z1p)
        zbt = exch((bb + offa, q), comm_z1.at[1],
                   z1_send.at[1], z1_recv.at[1], z1p)
        zt.start()
        zbt.start()
        rs_finish(1, 2, rb)
        rs_finish(3, 2, lb)
        zt.wait()
        z16[pl.ds(tb + offb, q), :] += comm_z1[0]
        zbt.wait()
        z16[pl.ds(bb + offb, q), :] += comm_z1[1]

        tb2 = tb + offb
        bb2 = bb + offb
        zt = exch((tb2 + (1 - bz2) * e, e), comm_z2.at[0],
                  z2_send.at[0], z2_recv.at[0], z2p)
        zbt = exch((bb2 + (1 - bz2) * e, e), comm_z2.at[1],
                   z2_send.at[1], z2_recv.at[1], z2p)
        zt.start()
        zbt.start()
        zt.wait()
        z16[pl.ds(tb2 + bz2 * e, e), :] += comm_z2[0]
        zbt.wait()
        z16[pl.ds(bb2 + bz2 * e, e), :] += comm_z2[1]

        tf = tb2 + bz2 * e
        bf = bb2 + bz2 * e
        for base in (tf, bf):
            zv = z16[pl.ds(base, e), :].astype(jnp.float32)
            sv = zv / (1.0 + jnp.exp(-zv))
            z16[pl.ds(base, e), :] = sv.astype(jnp.bfloat16)
            out_ref[pl.ds(base, e), :] = sv

        off0 = bz1 * q + bz2 * e
        off1 = bz1 * q + (1 - bz2) * e
        offL = (1 - bz1) * q

        def sub_info(kind, t):
            off, sz = ((off0, e), (off1, e), (offL, q))[kind % 3]
            if kind < 3:
                cs = lax.rem(w + 1 - t + PLANE, PLANE)
                cr_ = lax.rem(w - t + PLANE, PLANE)
                extra, peer = 0, p_right
            else:
                cs = lax.rem(w - 1 + t + PLANE, PLANE)
                cr_ = lax.rem(w + t, PLANE)
                extra, peer = half, p_left
            return cs * pch + extra + off, cr_ * pch + extra + off, sz, peer

        def sub_start(kind, t):
            sbase, _, sz, peer = sub_info(kind, t)
            rd = exch((sbase, sz), z16.at[pl.ds(sbase, sz), :],
                      ags.at[kind, t], agr.at[kind, t], peer)
            rd.start()
            return rd

        def conv(base, sz):
            out_ref[pl.ds(base, sz), :] = (
                z16[pl.ds(base, sz), :].astype(jnp.float32))

        def sub_finish(kind, t, rd):
            rd.wait()
            nxt = sub_start(kind, t + 1) if t + 1 < PLANE - 1 else None
            _, rbase, sz, _ = sub_info(kind, t)
            conv(rbase, sz)
            return nxt

        r0 = sub_start(0, 0)
        l0 = sub_start(3, 0)
        rd_t = exch((tf, e), z16.at[pl.ds(tf, e), :],
                    g2_send.at[0], g2_recv.at[0], z2p)
        rd_b = exch((bf, e), z16.at[pl.ds(bf, e), :],
                    g2_send.at[1], g2_recv.at[1], z2p)
        rd_t.start()
        rd_b.start()
        rd_t.wait()
        rd_b.wait()
        zd_t = exch((tb2, q), z16.at[pl.ds(tb2, q), :],
                    g1_send.at[0], g1_recv.at[0], z1p)
        zd_b = exch((bb2, q), z16.at[pl.ds(bb2, q), :],
                    g1_send.at[1], g1_recv.at[1], z1p)
        zd_t.start()
        zd_b.start()
        r1 = sub_start(1, 0)
        l1 = sub_start(4, 0)
        conv(tb2 + (1 - bz2) * e, e)
        conv(bb2 + (1 - bz2) * e, e)
        r0 = sub_finish(0, 0, r0)
        l0 = sub_finish(3, 0, l0)
        r1 = sub_finish(1, 0, r1)
        l1 = sub_finish(4, 0, l1)
        r0 = sub_finish(0, 1, r0)
        l0 = sub_finish(3, 1, l0)
        zd_t.wait()
        zd_b.wait()
        rL = sub_start(2, 0)
        lL = sub_start(5, 0)
        conv(tb + (1 - bz1) * q, q)
        conv(bb + (1 - bz1) * q, q)
        r1 = sub_finish(1, 1, r1)
        l1 = sub_finish(4, 1, l1)
        sub_finish(0, 2, r0)
        sub_finish(3, 2, l0)
        rL = sub_finish(2, 0, rL)
        lL = sub_finish(5, 0, lL)
        sub_finish(1, 2, r1)
        sub_finish(4, 2, l1)
        rL = sub_finish(2, 1, rL)
        lL = sub_finish(5, 1, lL)
        sub_finish(2, 2, rL)
        sub_finish(5, 2, lL)

    n_steps = PLANE - 1
    return pl.pallas_call(
        body,
        out_shape=jax.ShapeDtypeStruct((M, N), jnp.float32),
        in_specs=[
            pl.BlockSpec(memory_space=pltpu.VMEM),
            pl.BlockSpec(memory_space=pltpu.VMEM),
        ],
        out_specs=pl.BlockSpec(memory_space=pltpu.VMEM),
        scratch_shapes=[
            pltpu.VMEM((M, N), jnp.bfloat16),
            pltpu.VMEM((4, n_steps, q, N), jnp.bfloat16),
            pltpu.VMEM((2, q, N), jnp.bfloat16),
            pltpu.VMEM((2, e, N), jnp.bfloat16),
            pltpu.SemaphoreType.DMA((4, n_steps)),
            pltpu.SemaphoreType.DMA((4, n_steps)),
            pltpu.SemaphoreType.DMA((2,)),
            pltpu.SemaphoreType.DMA((2,)),
            pltpu.SemaphoreType.DMA((2,)),
            pltpu.SemaphoreType.DMA((2,)),
            pltpu.SemaphoreType.DMA((2,)),
            pltpu.SemaphoreType.DMA((2,)),
            pltpu.SemaphoreType.DMA((2,)),
            pltpu.SemaphoreType.DMA((2,)),
            pltpu.SemaphoreType.DMA((6, n_steps)),
            pltpu.SemaphoreType.DMA((6, n_steps)),
        ],
        compiler_params=pltpu.CompilerParams(
            collective_id=0,
            vmem_limit_bytes=100 * 1024 * 1024,
        ),
    )(A, B)


# device time: 79661 ns/iter; 3.3564x vs baseline; 1.0409x over previous
import jax
import jax.numpy as jnp
from jax import lax
from jax.experimental import pallas as pl
from jax.experimental.pallas import tpu as pltpu

N_DEV = 16
PLANE = 4


def kernel(A, B):
    M, K = A.shape
    _, N = B.shape
    pch = M // PLANE
    half = pch // 2
    q = half // 2
    e = q // 2
    nh = N // 2

    def body(a_ref, b_ref, out_ref, z16, rs_comm, comm_z1, comm_z2,
             ars, arr, z1s, z1r, z2s, z2r, g2s, g2r, g1s, g1r, ags, agr):
        my = lax.axis_index("i")
        w = lax.rem(my, PLANE)
        zz = lax.div(my, PLANE)
        zb = zz * PLANE
        p_right = zb + lax.rem(w + 1, PLANE)
        p_left = zb + lax.rem(w + PLANE - 1, PLANE)
        bz1 = lax.rem(zz, 2)
        bz2 = lax.div(zz, 2)
        z1p = my + 4 - 8 * bz1
        z2p = my + 8 - 16 * bz2

        barrier = pltpu.get_barrier_semaphore()
        for nbr in (p_left, p_right, z1p, z2p):
            pl.semaphore_signal(
                barrier, inc=1,
                device_id=(nbr,), device_id_type=pl.DeviceIdType.MESH,
            )
        pl.semaphore_wait(barrier, 4)

        def zslc(g, base, sz):
            return z16.at[pl.ds(base, sz), pl.ds(g * nh, nh)]

        def exch(g, rows, dst_ref, ssem, rsem, peer):
            return pltpu.make_async_remote_copy(
                src_ref=zslc(g, rows[0], rows[1]),
                dst_ref=dst_ref,
                send_sem=ssem,
                recv_sem=rsem,
                device_id=(peer,),
                device_id_type=pl.DeviceIdType.MESH,
            )

        def mm_slab(c):
            z16[pl.ds(c * pch, pch), :] = jnp.dot(
                a_ref[pl.ds(c * pch, pch), :], b_ref[:, :],
                preferred_element_type=jnp.float32,
            ).astype(jnp.bfloat16)

        offa = (1 - bz1) * q
        offb = bz1 * q

        def rs_info(kind, t):
            off = offa if kind % 2 == 0 else offb
            if kind < 2:
                cs = lax.rem(w - t + PLANE, PLANE)
                cr_ = lax.rem(w - t - 1 + PLANE, PLANE)
                extra, peer = 0, p_right
            else:
                cs = lax.rem(w + t, PLANE)
                cr_ = lax.rem(w + t + 1, PLANE)
                extra, peer = half, p_left
            return cs * pch + extra + off, cr_ * pch + extra + off, peer

        def rs_start(g, kind, t):
            sbase, _, peer = rs_info(kind, t)
            rd = exch(g, (sbase, q), rs_comm.at[g, kind, t],
                      ars.at[g, kind, t], arr.at[g, kind, t], peer)
            rd.start()
            return rd

        def rs_finish(g, kind, t, rd):
            rd.wait()
            _, rbase, _ = rs_info(kind, t)
            z16[pl.ds(rbase, q), pl.ds(g * nh, nh)] += rs_comm[g, kind, t]
            return rs_start(g, kind, t + 1) if t + 1 < PLANE - 1 else None

        def rs_starts(g):
            return [rs_start(g, 0, 0), rs_start(g, 2, 0),
                    rs_start(g, 1, 0), rs_start(g, 3, 0)]

        def rs_round(g, t, h):
            return [rs_finish(g, 0, t, h[0]), rs_finish(g, 2, t, h[1]),
                    rs_finish(g, 1, t, h[2]), rs_finish(g, 3, t, h[3])]

        tc = lax.rem(w + 1, PLANE)
        bc = lax.rem(w + PLANE - 1, PLANE)
        tb = tc * pch
        bb = bc * pch + half
        tb2 = tb + offb
        bb2 = bb + offb
        tf = tb2 + bz2 * e
        bf = bb2 + bz2 * e

        def z1_start(g):
            rt = exch(g, (tb + offa, q), comm_z1.at[g, 0],
                      z1s.at[g, 0], z1r.at[g, 0], z1p)
            rb = exch(g, (bb + offa, q), comm_z1.at[g, 1],
                      z1s.at[g, 1], z1r.at[g, 1], z1p)
            rt.start()
            rb.start()
            return rt, rb

        def z1_finish(g, h):
            h[0].wait()
            z16[pl.ds(tb2, q), pl.ds(g * nh, nh)] += comm_z1[g, 0]
            h[1].wait()
            z16[pl.ds(bb2, q), pl.ds(g * nh, nh)] += comm_z1[g, 1]

        def z2_start(g):
            rt = exch(g, (tb2 + (1 - bz2) * e, e), comm_z2.at[g, 0],
                      z2s.at[g, 0], z2r.at[g, 0], z2p)
            rb = exch(g, (bb2 + (1 - bz2) * e, e), comm_z2.at[g, 1],
                      z2s.at[g, 1], z2r.at[g, 1], z2p)
            rt.start()
            rb.start()
            return rt, rb

        def z2_finish(g, h):
            h[0].wait()
            z16[pl.ds(tf, e), pl.ds(g * nh, nh)] += comm_z2[g, 0]
            h[1].wait()
            z16[pl.ds(bf, e), pl.ds(g * nh, nh)] += comm_z2[g, 1]

        def silu(g):
            for base in (tf, bf):
                zv = z16[pl.ds(base, e), pl.ds(g * nh, nh)].astype(jnp.float32)
                sv = zv / (1.0 + jnp.exp(-zv))
                z16[pl.ds(base, e), pl.ds(g * nh, nh)] = sv.astype(jnp.bfloat16)
                out_ref[pl.ds(base, e), pl.ds(g * nh, nh)] = sv

        off0 = bz1 * q + bz2 * e
        off1 = bz1 * q + (1 - bz2) * e
        offL = (1 - bz1) * q

        def sub_info(kind, t):
            off, sz = ((off0, e), (off1, e), (offL, q))[kind % 3]
            if kind < 3:
                cs = lax.rem(w + 1 - t + PLANE, PLANE)
                cr_ = lax.rem(w - t + PLANE, PLANE)
                extra, peer = 0, p_right
            else:
                cs = lax.rem(w - 1 + t + PLANE, PLANE)
                cr_ = lax.rem(w + t, PLANE)
                extra, peer = half, p_left
            return cs * pch + extra + off, cr_ * pch + extra + off, sz, peer

        def sub_start(g, kind, t):
            sbase, _, sz, peer = sub_info(kind, t)
            rd = exch(g, (sbase, sz), zslc(g, sbase, sz),
                      ags.at[g, kind, t], agr.at[g, kind, t], peer)
            rd.start()
            return rd

        def conv(g, base, sz):
            out_ref[pl.ds(base, sz), pl.ds(g * nh, nh)] = (
                z16[pl.ds(base, sz), pl.ds(g * nh, nh)].astype(jnp.float32))

        def sub_finish(g, kind, t, rd):
            rd.wait()
            nxt = sub_start(g, kind, t + 1) if t + 1 < PLANE - 1 else None
            _, rbase, sz, _ = sub_info(kind, t)
            conv(g, rbase, sz)
            return nxt

        def ag_open(g):
            r0 = sub_start(g, 0, 0)
            l0 = sub_start(g, 3, 0)
            t2 = exch(g, (tf, e), zslc(g, tf, e),
                      g2s.at[g, 0], g2r.at[g, 0], z2p)
            b2 = exch(g, (bf, e), zslc(g, bf, e),
                      g2s.at[g, 1], g2r.at[g, 1], z2p)
            t2.start()
            b2.start()
            return [r0, l0, t2, b2]

        def ag_mid(g, st):
            r0, l0, t2, b2 = st
            t2.wait()
            b2.wait()
            t1 = exch(g, (tb2, q), zslc(g, tb2, q),
                      g1s.at[g, 0], g1r.at[g, 0], z1p)
            b1 = exch(g, (bb2, q), zslc(g, bb2, q),
                      g1s.at[g, 1], g1r.at[g, 1], z1p)
            t1.start()
            b1.start()
            r1 = sub_start(g, 1, 0)
            l1 = sub_start(g, 4, 0)
            conv(g, tb2 + (1 - bz2) * e, e)
            conv(g, bb2 + (1 - bz2) * e, e)
            r0 = sub_finish(g, 0, 0, r0)
            l0 = sub_finish(g, 3, 0, l0)
            r1 = sub_finish(g, 1, 0, r1)
            l1 = sub_finish(g, 4, 0, l1)
            r0 = sub_finish(g, 0, 1, r0)
            l0 = sub_finish(g, 3, 1, l0)
            return [r0, l0, r1, l1, t1, b1]

        def ag_close(g, st):
            r0, l0, r1, l1, t1, b1 = st
            t1.wait()
            b1.wait()
            rL = sub_start(g, 2, 0)
            lL = sub_start(g, 5, 0)
            conv(g, tb + offa, q)
            conv(g, bb + offa, q)
            r1 = sub_finish(g, 1, 1, r1)
            l1 = sub_finish(g, 4, 1, l1)
            sub_finish(g, 0, 2, r0)
            sub_finish(g, 3, 2, l0)
            rL = sub_finish(g, 2, 0, rL)
            lL = sub_finish(g, 5, 0, lL)
            sub_finish(g, 1, 2, r1)
            sub_finish(g, 4, 2, l1)
            rL = sub_finish(g, 2, 1, rL)
            lL = sub_finish(g, 5, 1, lL)
            sub_finish(g, 2, 2, rL)
            sub_finish(g, 5, 2, lL)

        mm_slab(w)
        ha = rs_starts(0)
        mm_slab(lax.rem(w + PLANE - 1, PLANE))
        mm_slab(lax.rem(w + 1, PLANE))
        mm_slab(lax.rem(w + 2, PLANE))
        ha = rs_round(0, 0, ha)
        ha = rs_round(0, 1, ha)
        hb = rs_starts(1)
        rs_finish(0, 0, 2, ha[0])
        rs_finish(0, 2, 2, ha[1])
        az1 = z1_start(0)
        rs_finish(0, 1, 2, ha[2])
        rs_finish(0, 3, 2, ha[3])
        hb = rs_round(1, 0, hb)
        z1_finish(0, az1)
        az2 = z2_start(0)
        hb = rs_round(1, 1, hb)
        z2_finish(0, az2)
        silu(0)
        ag_a = ag_open(0)
        rs_finish(1, 0, 2, hb[0])
        rs_finish(1, 2, 2, hb[1])
        bz1h = z1_start(1)
        rs_finish(1, 1, 2, hb[2])
        rs_finish(1, 3, 2, hb[3])
        ag_a = ag_mid(0, ag_a)
        z1_finish(1, bz1h)
        bz2h = z2_start(1)
        ag_close(0, ag_a)
        z2_finish(1, bz2h)
        silu(1)
        ag_b = ag_open(1)
        ag_b = ag_mid(1, ag_b)
        ag_close(1, ag_b)

    n_steps = PLANE - 1
    return pl.pallas_call(
        body,
        out_shape=jax.ShapeDtypeStruct((M, N), jnp.float32),
        in_specs=[
            pl.BlockSpec(memory_space=pltpu.VMEM),
            pl.BlockSpec(memory_space=pltpu.VMEM),
        ],
        out_specs=pl.BlockSpec(memory_space=pltpu.VMEM),
        scratch_shapes=[
            pltpu.VMEM((M, N), jnp.bfloat16),
            pltpu.VMEM((2, 4, n_steps, q, nh), jnp.bfloat16),
            pltpu.VMEM((2, 2, q, nh), jnp.bfloat16),
            pltpu.VMEM((2, 2, e, nh), jnp.bfloat16),
            pltpu.SemaphoreType.DMA((2, 4, n_steps)),
            pltpu.SemaphoreType.DMA((2, 4, n_steps)),
            pltpu.SemaphoreType.DMA((2, 2)),
            pltpu.SemaphoreType.DMA((2, 2)),
            pltpu.SemaphoreType.DMA((2, 2)),
            pltpu.SemaphoreType.DMA((2, 2)),
            pltpu.SemaphoreType.DMA((2, 2)),
            pltpu.SemaphoreType.DMA((2, 2)),
            pltpu.SemaphoreType.DMA((2, 2)),
            pltpu.SemaphoreType.DMA((2, 2)),
            pltpu.SemaphoreType.DMA((2, 6, n_steps)),
            pltpu.SemaphoreType.DMA((2, 6, n_steps)),
        ],
        compiler_params=pltpu.CompilerParams(
            collective_id=0,
            vmem_limit_bytes=100 * 1024 * 1024,
        ),
    )(A, B)


# device time: 79301 ns/iter; 3.3716x vs baseline; 1.0045x over previous
import jax
import jax.numpy as jnp
from jax import lax
from jax.experimental import pallas as pl
from jax.experimental.pallas import tpu as pltpu

N_DEV = 16
PLANE = 4


def kernel(A, B):
    M, K = A.shape
    _, N = B.shape
    pch = M // PLANE
    half = pch // 2
    q = half // 2
    e = q // 2
    nh = N // 2

    def body(a_ref, b_ref, out_ref, z16, rs_comm, comm_z1, comm_z2,
             ars, arr, z1s, z1r, z2s, z2r, g2s, g2r, g1s, g1r, ags, agr):
        my = lax.axis_index("i")
        w = lax.rem(my, PLANE)
        zz = lax.div(my, PLANE)
        zb = zz * PLANE
        p_right = zb + lax.rem(w + 1, PLANE)
        p_left = zb + lax.rem(w + PLANE - 1, PLANE)
        bz1 = lax.rem(zz, 2)
        bz2 = lax.div(zz, 2)
        z1p = my + 4 - 8 * bz1
        z2p = my + 8 - 16 * bz2

        barrier = pltpu.get_barrier_semaphore()
        for nbr in (p_left, p_right, z1p, z2p):
            pl.semaphore_signal(
                barrier, inc=1,
                device_id=(nbr,), device_id_type=pl.DeviceIdType.MESH,
            )
        pl.semaphore_wait(barrier, 4)

        def zslc(g, base, sz):
            return z16.at[pl.ds(base, sz), pl.ds(g * nh, nh)]

        def exch(g, rows, dst_ref, ssem, rsem, peer):
            return pltpu.make_async_remote_copy(
                src_ref=zslc(g, rows[0], rows[1]),
                dst_ref=dst_ref,
                send_sem=ssem,
                recv_sem=rsem,
                device_id=(peer,),
                device_id_type=pl.DeviceIdType.MESH,
            )

        def mm_slab(c):
            z16[pl.ds(c * pch, pch), :] = jnp.dot(
                a_ref[pl.ds(c * pch, pch), :], b_ref[:, :],
                preferred_element_type=jnp.float32,
            ).astype(jnp.bfloat16)

        offa = (1 - bz1) * q
        offb = bz1 * q

        def rs_info(kind, t):
            off = offa if kind % 2 == 0 else offb
            if kind < 2:
                cs = lax.rem(w - t + PLANE, PLANE)
                cr_ = lax.rem(w - t - 1 + PLANE, PLANE)
                extra, peer = 0, p_right
            else:
                cs = lax.rem(w + t, PLANE)
                cr_ = lax.rem(w + t + 1, PLANE)
                extra, peer = half, p_left
            return cs * pch + extra + off, cr_ * pch + extra + off, peer

        def rs_start(g, kind, t):
            sbase, _, peer = rs_info(kind, t)
            rd = exch(g, (sbase, q), rs_comm.at[g, kind, t],
                      ars.at[g, kind, t], arr.at[g, kind, t], peer)
            rd.start()
            return rd

        def rs_finish(g, kind, t, rd):
            rd.wait()
            _, rbase, _ = rs_info(kind, t)
            z16[pl.ds(rbase, q), pl.ds(g * nh, nh)] += rs_comm[g, kind, t]
            return rs_start(g, kind, t + 1) if t + 1 < PLANE - 1 else None

        def rs_starts(g):
            return [rs_start(g, 0, 0), rs_start(g, 2, 0),
                    rs_start(g, 1, 0), rs_start(g, 3, 0)]

        def rs_round(g, t, h):
            return [rs_finish(g, 0, t, h[0]), rs_finish(g, 2, t, h[1]),
                    rs_finish(g, 1, t, h[2]), rs_finish(g, 3, t, h[3])]

        tc = lax.rem(w + 1, PLANE)
        bc = lax.rem(w + PLANE - 1, PLANE)
        tb = tc * pch
        bb = bc * pch + half
        tb2 = tb + offb
        bb2 = bb + offb
        tf = tb2 + bz2 * e
        bf = bb2 + bz2 * e

        def z1_start(g):
            rt = exch(g, (tb + offa, q), comm_z1.at[g, 0],
                      z1s.at[g, 0], z1r.at[g, 0], z1p)
            rb = exch(g, (bb + offa, q), comm_z1.at[g, 1],
                      z1s.at[g, 1], z1r.at[g, 1], z1p)
            rt.start()
            rb.start()
            return rt, rb

        def z1_finish(g, h):
            h[0].wait()
            z16[pl.ds(tb2, q), pl.ds(g * nh, nh)] += comm_z1[g, 0]
            h[1].wait()
            z16[pl.ds(bb2, q), pl.ds(g * nh, nh)] += comm_z1[g, 1]

        def z2_start(g):
            rt = exch(g, (tb2 + (1 - bz2) * e, e), comm_z2.at[g, 0],
                      z2s.at[g, 0], z2r.at[g, 0], z2p)
            rb = exch(g, (bb2 + (1 - bz2) * e, e), comm_z2.at[g, 1],
                      z2s.at[g, 1], z2r.at[g, 1], z2p)
            rt.start()
            rb.start()
            return rt, rb

        def z2_finish(g, h):
            h[0].wait()
            z16[pl.ds(tf, e), pl.ds(g * nh, nh)] += comm_z2[g, 0]
            h[1].wait()
            z16[pl.ds(bf, e), pl.ds(g * nh, nh)] += comm_z2[g, 1]

        def silu(g):
            for base in (tf, bf):
                zv = z16[pl.ds(base, e), pl.ds(g * nh, nh)].astype(jnp.float32)
                sv = zv / (1.0 + jnp.exp(-zv))
                z16[pl.ds(base, e), pl.ds(g * nh, nh)] = sv.astype(jnp.bfloat16)
                out_ref[pl.ds(base, e), pl.ds(g * nh, nh)] = sv

        off0 = bz1 * q + bz2 * e
        off1 = bz1 * q + (1 - bz2) * e
        offL = (1 - bz1) * q

        def sub_info(kind, t):
            off, sz = ((off0, e), (off1, e), (offL, q))[kind % 3]
            if kind < 3:
                cs = lax.rem(w + 1 - t + PLANE, PLANE)
                cr_ = lax.rem(w - t + PLANE, PLANE)
                extra, peer = 0, p_right
            else:
                cs = lax.rem(w - 1 + t + PLANE, PLANE)
                cr_ = lax.rem(w + t, PLANE)
                extra, peer = half, p_left
            return cs * pch + extra + off, cr_ * pch + extra + off, sz, peer

        def sub_start(g, kind, t):
            sbase, _, sz, peer = sub_info(kind, t)
            rd = exch(g, (sbase, sz), zslc(g, sbase, sz),
                      ags.at[g, kind, t], agr.at[g, kind, t], peer)
            rd.start()
            return rd

        def conv(g, base, sz):
            out_ref[pl.ds(base, sz), pl.ds(g * nh, nh)] = (
                z16[pl.ds(base, sz), pl.ds(g * nh, nh)].astype(jnp.float32))

        def sub_finish(g, kind, t, rd):
            rd.wait()
            nxt = sub_start(g, kind, t + 1) if t + 1 < PLANE - 1 else None
            _, rbase, sz, _ = sub_info(kind, t)
            conv(g, rbase, sz)
            return nxt

        def ag_open(g):
            r0 = sub_start(g, 0, 0)
            l0 = sub_start(g, 3, 0)
            t2 = exch(g, (tf, e), zslc(g, tf, e),
                      g2s.at[g, 0], g2r.at[g, 0], z2p)
            b2 = exch(g, (bf, e), zslc(g, bf, e),
                      g2s.at[g, 1], g2r.at[g, 1], z2p)
            t2.start()
            b2.start()
            return [r0, l0, t2, b2]

        def ag_mid(g, st):
            r0, l0, t2, b2 = st
            t2.wait()
            b2.wait()
            t1 = exch(g, (tb2, q), zslc(g, tb2, q),
                      g1s.at[g, 0], g1r.at[g, 0], z1p)
            b1 = exch(g, (bb2, q), zslc(g, bb2, q),
                      g1s.at[g, 1], g1r.at[g, 1], z1p)
            t1.start()
            b1.start()
            r1 = sub_start(g, 1, 0)
            l1 = sub_start(g, 4, 0)
            conv(g, tb2 + (1 - bz2) * e, e)
            conv(g, bb2 + (1 - bz2) * e, e)
            r0 = sub_finish(g, 0, 0, r0)
            l0 = sub_finish(g, 3, 0, l0)
            r1 = sub_finish(g, 1, 0, r1)
            l1 = sub_finish(g, 4, 0, l1)
            r0 = sub_finish(g, 0, 1, r0)
            l0 = sub_finish(g, 3, 1, l0)
            return [r0, l0, r1, l1, t1, b1]

        def ag_close(g, st):
            r0, l0, r1, l1, t1, b1 = st
            t1.wait()
            b1.wait()
            rL = sub_start(g, 2, 0)
            lL = sub_start(g, 5, 0)
            conv(g, tb + offa, q)
            conv(g, bb + offa, q)
            r1 = sub_finish(g, 1, 1, r1)
            l1 = sub_finish(g, 4, 1, l1)
            sub_finish(g, 0, 2, r0)
            sub_finish(g, 3, 2, l0)
            rL = sub_finish(g, 2, 0, rL)
            lL = sub_finish(g, 5, 0, lL)
            sub_finish(g, 1, 2, r1)
            sub_finish(g, 4, 2, l1)
            rL = sub_finish(g, 2, 1, rL)
            lL = sub_finish(g, 5, 1, lL)
            sub_finish(g, 2, 2, rL)
            sub_finish(g, 5, 2, lL)

        mm_slab(w)
        ha = rs_starts(0)
        mm_slab(lax.rem(w + PLANE - 1, PLANE))
        mm_slab(lax.rem(w + 1, PLANE))
        mm_slab(lax.rem(w + 2, PLANE))
        ha = rs_round(0, 0, ha)
        hb = rs_starts(1)
        ha = rs_round(0, 1, ha)
        hb = rs_round(1, 0, hb)
        rs_finish(0, 0, 2, ha[0])
        rs_finish(0, 2, 2, ha[1])
        az1 = z1_start(0)
        rs_finish(0, 1, 2, ha[2])
        rs_finish(0, 3, 2, ha[3])
        hb = rs_round(1, 1, hb)
        z1_finish(0, az1)
        az2 = z2_start(0)
        rs_finish(1, 0, 2, hb[0])
        rs_finish(1, 2, 2, hb[1])
        bz1h = z1_start(1)
        rs_finish(1, 1, 2, hb[2])
        rs_finish(1, 3, 2, hb[3])
        z2_finish(0, az2)
        silu(0)
        ag_a = ag_open(0)
        z1_finish(1, bz1h)
        bz2h = z2_start(1)
        ag_a = ag_mid(0, ag_a)
        z2_finish(1, bz2h)
        silu(1)
        ag_b = ag_open(1)
        ag_close(0, ag_a)
        ag_b = ag_mid(1, ag_b)
        ag_close(1, ag_b)

    n_steps = PLANE - 1
    return pl.pallas_call(
        body,
        out_shape=jax.ShapeDtypeStruct((M, N), jnp.float32),
        in_specs=[
            pl.BlockSpec(memory_space=pltpu.VMEM),
            pl.BlockSpec(memory_space=pltpu.VMEM),
        ],
        out_specs=pl.BlockSpec(memory_space=pltpu.VMEM),
        scratch_shapes=[
            pltpu.VMEM((M, N), jnp.bfloat16),
            pltpu.VMEM((2, 4, n_steps, q, nh), jnp.bfloat16),
            pltpu.VMEM((2, 2, q, nh), jnp.bfloat16),
            pltpu.VMEM((2, 2, e, nh), jnp.bfloat16),
            pltpu.SemaphoreType.DMA((2, 4, n_steps)),
            pltpu.SemaphoreType.DMA((2, 4, n_steps)),
            pltpu.SemaphoreType.DMA((2, 2)),
            pltpu.SemaphoreType.DMA((2, 2)),
            pltpu.SemaphoreType.DMA((2, 2)),
            pltpu.SemaphoreType.DMA((2, 2)),
            pltpu.SemaphoreType.DMA((2, 2)),
            pltpu.SemaphoreType.DMA((2, 2)),
            pltpu.SemaphoreType.DMA((2, 2)),
            pltpu.SemaphoreType.DMA((2, 2)),
            pltpu.SemaphoreType.DMA((2, 6, n_steps)),
            pltpu.SemaphoreType.DMA((2, 6, n_steps)),
        ],
        compiler_params=pltpu.CompilerParams(
            collective_id=0,
            vmem_limit_bytes=100 * 1024 * 1024,
        ),
    )(A, B)


# device time: 78105 ns/iter; 3.4232x vs baseline; 1.0153x over previous
import jax
import jax.numpy as jnp
from jax import lax
from jax.experimental import pallas as pl
from jax.experimental.pallas import tpu as pltpu

N_DEV = 16
PLANE = 4


def kernel(A, B):
    M, K = A.shape
    _, N = B.shape
    pch = M // PLANE
    half = pch // 2
    q = half // 2
    e = q // 2
    nh = N // 2

    def body(a_ref, b_ref, out_ref, z16, rs_comm, comm_z1, comm_z2,
             ars, arr, z1s, z1r, z2s, z2r, g2s, g2r, g1s, g1r, ags, agr):
        my = lax.axis_index("i")
        w = lax.rem(my, PLANE)
        zz = lax.div(my, PLANE)
        zb = zz * PLANE
        p_right = zb + lax.rem(w + 1, PLANE)
        p_left = zb + lax.rem(w + PLANE - 1, PLANE)
        bz1 = lax.rem(zz, 2)
        bz2 = lax.div(zz, 2)
        z1p = my + 4 - 8 * bz1
        z2p = my + 8 - 16 * bz2

        barrier = pltpu.get_barrier_semaphore()
        for nbr in (p_left, p_right, z1p, z2p):
            pl.semaphore_signal(
                barrier, inc=1,
                device_id=(nbr,), device_id_type=pl.DeviceIdType.MESH,
            )

        def zslc(g, base, sz):
            return z16.at[pl.ds(base, sz), pl.ds(g * nh, nh)]

        def exch(g, rows, dst_ref, ssem, rsem, peer):
            return pltpu.make_async_remote_copy(
                src_ref=zslc(g, rows[0], rows[1]),
                dst_ref=dst_ref,
                send_sem=ssem,
                recv_sem=rsem,
                device_id=(peer,),
                device_id_type=pl.DeviceIdType.MESH,
            )

        def mm_slab(c):
            z16[pl.ds(c * pch, pch), :] = jnp.dot(
                a_ref[pl.ds(c * pch, pch), :].astype(jnp.bfloat16),
                b_ref[:, :].astype(jnp.bfloat16),
                preferred_element_type=jnp.float32,
            ).astype(jnp.bfloat16)

        offa = (1 - bz1) * q
        offb = bz1 * q

        def rs_info(kind, t):
            off = offa if kind % 2 == 0 else offb
            if kind < 2:
                cs = lax.rem(w - t + PLANE, PLANE)
                cr_ = lax.rem(w - t - 1 + PLANE, PLANE)
                extra, peer = 0, p_right
            else:
                cs = lax.rem(w + t, PLANE)
                cr_ = lax.rem(w + t + 1, PLANE)
                extra, peer = half, p_left
            return cs * pch + extra + off, cr_ * pch + extra + off, peer

        def rs_start(g, kind, t):
            sbase, _, peer = rs_info(kind, t)
            rd = exch(g, (sbase, q), rs_comm.at[g, kind, t],
                      ars.at[g, kind, t], arr.at[g, kind, t], peer)
            rd.start()
            return rd

        def rs_finish(g, kind, t, rd):
            rd.wait()
            _, rbase, _ = rs_info(kind, t)
            z16[pl.ds(rbase, q), pl.ds(g * nh, nh)] += rs_comm[g, kind, t]
            return rs_start(g, kind, t + 1) if t + 1 < PLANE - 1 else None

        def rs_starts(g):
            return [rs_start(g, 0, 0), rs_start(g, 2, 0),
                    rs_start(g, 1, 0), rs_start(g, 3, 0)]

        def rs_round(g, t, h):
            return [rs_finish(g, 0, t, h[0]), rs_finish(g, 2, t, h[1]),
                    rs_finish(g, 1, t, h[2]), rs_finish(g, 3, t, h[3])]

        tc = lax.rem(w + 1, PLANE)
        bc = lax.rem(w + PLANE - 1, PLANE)
        tb = tc * pch
        bb = bc * pch + half
        tb2 = tb + offb
        bb2 = bb + offb
        tf = tb2 + bz2 * e
        bf = bb2 + bz2 * e

        def z1_start(g):
            rt = exch(g, (tb + offa, q), comm_z1.at[g, 0],
                      z1s.at[g, 0], z1r.at[g, 0], z1p)
            rb = exch(g, (bb + offa, q), comm_z1.at[g, 1],
                      z1s.at[g, 1], z1r.at[g, 1], z1p)
            rt.start()
            rb.start()
            return rt, rb

        def z1_finish(g, h):
            h[0].wait()
            z16[pl.ds(tb2, q), pl.ds(g * nh, nh)] += comm_z1[g, 0]
            h[1].wait()
            z16[pl.ds(bb2, q), pl.ds(g * nh, nh)] += comm_z1[g, 1]

        def z2_start(g):
            rt = exch(g, (tb2 + (1 - bz2) * e, e), comm_z2.at[g, 0],
                      z2s.at[g, 0], z2r.at[g, 0], z2p)
            rb = exch(g, (bb2 + (1 - bz2) * e, e), comm_z2.at[g, 1],
                      z2s.at[g, 1], z2r.at[g, 1], z2p)
            rt.start()
            rb.start()
            return rt, rb

        def z2_finish(g, h):
            h[0].wait()
            z16[pl.ds(tf, e), pl.ds(g * nh, nh)] += comm_z2[g, 0]
            h[1].wait()
            z16[pl.ds(bf, e), pl.ds(g * nh, nh)] += comm_z2[g, 1]

        def silu(g):
            for base in (tf, bf):
                zv = z16[pl.ds(base, e), pl.ds(g * nh, nh)].astype(jnp.float32)
                sv = zv / (1.0 + jnp.exp(-zv))
                z16[pl.ds(base, e), pl.ds(g * nh, nh)] = sv.astype(jnp.bfloat16)
                out_ref[pl.ds(base, e), pl.ds(g * nh, nh)] = sv

        off0 = bz1 * q + bz2 * e
        off1 = bz1 * q + (1 - bz2) * e
        offL = (1 - bz1) * q

        def sub_info(kind, t):
            off, sz = ((off0, e), (off1, e), (offL, q))[kind % 3]
            if kind < 3:
                cs = lax.rem(w + 1 - t + PLANE, PLANE)
                cr_ = lax.rem(w - t + PLANE, PLANE)
                extra, peer = 0, p_right
            else:
                cs = lax.rem(w - 1 + t + PLANE, PLANE)
                cr_ = lax.rem(w + t, PLANE)
                extra, peer = half, p_left
            return cs * pch + extra + off, cr_ * pch + extra + off, sz, peer

        def sub_start(g, kind, t):
            sbase, _, sz, peer = sub_info(kind, t)
            rd = exch(g, (sbase, sz), zslc(g, sbase, sz),
                      ags.at[g, kind, t], agr.at[g, kind, t], peer)
            rd.start()
            return rd

        def conv(g, base, sz):
            out_ref[pl.ds(base, sz), pl.ds(g * nh, nh)] = (
                z16[pl.ds(base, sz), pl.ds(g * nh, nh)].astype(jnp.float32))

        def sub_finish(g, kind, t, rd):
            rd.wait()
            nxt = sub_start(g, kind, t + 1) if t + 1 < PLANE - 1 else None
            _, rbase, sz, _ = sub_info(kind, t)
            conv(g, rbase, sz)
            return nxt

        def ag_open(g):
            r0 = sub_start(g, 0, 0)
            l0 = sub_start(g, 3, 0)
            t2 = exch(g, (tf, e), zslc(g, tf, e),
                      g2s.at[g, 0], g2r.at[g, 0], z2p)
            b2 = exch(g, (bf, e), zslc(g, bf, e),
                      g2s.at[g, 1], g2r.at[g, 1], z2p)
            t2.start()
            b2.start()
            return [r0, l0, t2, b2]

        def ag_mid(g, st):
            r0, l0, t2, b2 = st
            t2.wait()
            b2.wait()
            t1 = exch(g, (tb2, q), zslc(g, tb2, q),
                      g1s.at[g, 0], g1r.at[g, 0], z1p)
            b1 = exch(g, (bb2, q), zslc(g, bb2, q),
                      g1s.at[g, 1], g1r.at[g, 1], z1p)
            t1.start()
            b1.start()
            r1 = sub_start(g, 1, 0)
            l1 = sub_start(g, 4, 0)
            conv(g, tb2 + (1 - bz2) * e, e)
            conv(g, bb2 + (1 - bz2) * e, e)
            r0 = sub_finish(g, 0, 0, r0)
            l0 = sub_finish(g, 3, 0, l0)
            r1 = sub_finish(g, 1, 0, r1)
            l1 = sub_finish(g, 4, 0, l1)
            r0 = sub_finish(g, 0, 1, r0)
            l0 = sub_finish(g, 3, 1, l0)
            return [r0, l0, r1, l1, t1, b1]

        def ag_close(g, st):
            r0, l0, r1, l1, t1, b1 = st
            t1.wait()
            b1.wait()
            rL = sub_start(g, 2, 0)
            lL = sub_start(g, 5, 0)
            conv(g, tb + offa, q)
            conv(g, bb + offa, q)
            r1 = sub_finish(g, 1, 1, r1)
            l1 = sub_finish(g, 4, 1, l1)
            sub_finish(g, 0, 2, r0)
            sub_finish(g, 3, 2, l0)
            rL = sub_finish(g, 2, 0, rL)
            lL = sub_finish(g, 5, 0, lL)
            sub_finish(g, 1, 2, r1)
            sub_finish(g, 4, 2, l1)
            rL = sub_finish(g, 2, 1, rL)
            lL = sub_finish(g, 5, 1, lL)
            sub_finish(g, 2, 2, rL)
            sub_finish(g, 5, 2, lL)

        mm_slab(w)
        pl.semaphore_wait(barrier, 4)
        ha = rs_starts(0)
        mm_slab(lax.rem(w + PLANE - 1, PLANE))
        mm_slab(lax.rem(w + 1, PLANE))
        mm_slab(lax.rem(w + 2, PLANE))
        ha = rs_round(0, 0, ha)
        hb = rs_starts(1)
        ha = rs_round(0, 1, ha)
        hb = rs_round(1, 0, hb)
        rs_finish(0, 0, 2, ha[0])
        rs_finish(0, 2, 2, ha[1])
        az1 = z1_start(0)
        rs_finish(0, 1, 2, ha[2])
        rs_finish(0, 3, 2, ha[3])
        hb = rs_round(1, 1, hb)
        z1_finish(0, az1)
        az2 = z2_start(0)
        rs_finish(1, 0, 2, hb[0])
        rs_finish(1, 2, 2, hb[1])
        bz1h = z1_start(1)
        rs_finish(1, 1, 2, hb[2])
        rs_finish(1, 3, 2, hb[3])
        z2_finish(0, az2)
        silu(0)
        ag_a = ag_open(0)
        z1_finish(1, bz1h)
        bz2h = z2_start(1)
        ag_a = ag_mid(0, ag_a)
        z2_finish(1, bz2h)
        silu(1)
        ag_b = ag_open(1)
        ag_close(0, ag_a)
        ag_b = ag_mid(1, ag_b)
        ag_close(1, ag_b)

    n_steps = PLANE - 1
    return pl.pallas_call(
        body,
        out_shape=jax.ShapeDtypeStruct((M, N), jnp.float32),
        in_specs=[
            pl.BlockSpec(memory_space=pltpu.VMEM),
            pl.BlockSpec(memory_space=pltpu.VMEM),
        ],
        out_specs=pl.BlockSpec(memory_space=pltpu.VMEM),
        scratch_shapes=[
            pltpu.VMEM((M, N), jnp.bfloat16),
            pltpu.VMEM((2, 4, n_steps, q, nh), jnp.bfloat16),
            pltpu.VMEM((2, 2, q, nh), jnp.bfloat16),
            pltpu.VMEM((2, 2, e, nh), jnp.bfloat16),
            pltpu.SemaphoreType.DMA((2, 4, n_steps)),
            pltpu.SemaphoreType.DMA((2, 4, n_steps)),
            pltpu.SemaphoreType.DMA((2, 2)),
            pltpu.SemaphoreType.DMA((2, 2)),
            pltpu.SemaphoreType.DMA((2, 2)),
            pltpu.SemaphoreType.DMA((2, 2)),
            pltpu.SemaphoreType.DMA((2, 2)),
            pltpu.SemaphoreType.DMA((2, 2)),
            pltpu.SemaphoreType.DMA((2, 2)),
            pltpu.SemaphoreType.DMA((2, 2)),
            pltpu.SemaphoreType.DMA((2, 6, n_steps)),
            pltpu.SemaphoreType.DMA((2, 6, n_steps)),
        ],
        compiler_params=pltpu.CompilerParams(
            collective_id=0,
            vmem_limit_bytes=100 * 1024 * 1024,
        ),
    )(A, B)


# device time: 74501 ns/iter; 3.5888x vs baseline; 1.0484x over previous
import jax
import jax.numpy as jnp
from jax import lax
from jax.experimental import pallas as pl
from jax.experimental.pallas import tpu as pltpu

N_DEV = 16
PLANE = 4


def kernel(A, B):
    M, K = A.shape
    _, N = B.shape
    pch = M // PLANE
    half = pch // 2
    q = half // 2
    e = q // 2
    nh = N // 2

    def body(a_ref, b_ref, out_ref, b16, rs_comm, comm_z1, comm_z2,
             ars, arr, z1s, z1r, z2s, z2r, g2s, g2r, g1s, g1r, ags, agr):
        my = lax.axis_index("i")
        w = lax.rem(my, PLANE)
        zz = lax.div(my, PLANE)
        zb = zz * PLANE
        p_right = zb + lax.rem(w + 1, PLANE)
        p_left = zb + lax.rem(w + PLANE - 1, PLANE)
        bz1 = lax.rem(zz, 2)
        bz2 = lax.div(zz, 2)
        z1p = my + 4 - 8 * bz1
        z2p = my + 8 - 16 * bz2

        barrier = pltpu.get_barrier_semaphore()
        for nbr in (p_left, p_right, z1p, z2p):
            pl.semaphore_signal(
                barrier, inc=1,
                device_id=(nbr,), device_id_type=pl.DeviceIdType.MESH,
            )

        def zslc(g, base, sz):
            return out_ref.at[pl.ds(base, sz), pl.ds(g * nh, nh)]

        def exch(g, rows, dst_ref, ssem, rsem, peer):
            return pltpu.make_async_remote_copy(
                src_ref=zslc(g, rows[0], rows[1]),
                dst_ref=dst_ref,
                send_sem=ssem,
                recv_sem=rsem,
                device_id=(peer,),
                device_id_type=pl.DeviceIdType.MESH,
            )

        b16[:, :] = b_ref[:, :].astype(jnp.bfloat16)

        def mm_slab(c):
            out_ref[pl.ds(c * pch, pch), :] = jnp.dot(
                a_ref[pl.ds(c * pch, pch), :].astype(jnp.bfloat16),
                b16[:, :],
                preferred_element_type=jnp.float32,
            ).astype(jnp.bfloat16)

        offa = (1 - bz1) * q
        offb = bz1 * q

        def rs_info(kind, t):
            off = offa if kind % 2 == 0 else offb
            if kind < 2:
                cs = lax.rem(w - t + PLANE, PLANE)
                cr_ = lax.rem(w - t - 1 + PLANE, PLANE)
                extra, peer = 0, p_right
            else:
                cs = lax.rem(w + t, PLANE)
                cr_ = lax.rem(w + t + 1, PLANE)
                extra, peer = half, p_left
            return cs * pch + extra + off, cr_ * pch + extra + off, peer

        def rs_start(g, kind, t):
            sbase, _, peer = rs_info(kind, t)
            rd = exch(g, (sbase, q), rs_comm.at[g, kind, t],
                      ars.at[g, kind, t], arr.at[g, kind, t], peer)
            rd.start()
            return rd

        def rs_finish(g, kind, t, rd):
            rd.wait()
            _, rbase, _ = rs_info(kind, t)
            out_ref[pl.ds(rbase, q), pl.ds(g * nh, nh)] += rs_comm[g, kind, t]
            return rs_start(g, kind, t + 1) if t + 1 < PLANE - 1 else None

        def rs_starts(g):
            return [rs_start(g, 0, 0), rs_start(g, 2, 0),
                    rs_start(g, 1, 0), rs_start(g, 3, 0)]

        def rs_round(g, t, h):
            return [rs_finish(g, 0, t, h[0]), rs_finish(g, 2, t, h[1]),
                    rs_finish(g, 1, t, h[2]), rs_finish(g, 3, t, h[3])]

        tc = lax.rem(w + 1, PLANE)
        bc = lax.rem(w + PLANE - 1, PLANE)
        tb = tc * pch
        bb = bc * pch + half
        tb2 = tb + offb
        bb2 = bb + offb
        tf = tb2 + bz2 * e
        bf = bb2 + bz2 * e

        def z1_start(g):
            rt = exch(g, (tb + offa, q), comm_z1.at[g, 0],
                      z1s.at[g, 0], z1r.at[g, 0], z1p)
            rb = exch(g, (bb + offa, q), comm_z1.at[g, 1],
                      z1s.at[g, 1], z1r.at[g, 1], z1p)
            rt.start()
            rb.start()
            return rt, rb

        def z1_finish(g, h):
            h[0].wait()
            out_ref[pl.ds(tb2, q), pl.ds(g * nh, nh)] += comm_z1[g, 0]
            h[1].wait()
            out_ref[pl.ds(bb2, q), pl.ds(g * nh, nh)] += comm_z1[g, 1]

        def z2_start(g):
            rt = exch(g, (tb2 + (1 - bz2) * e, e), comm_z2.at[g, 0],
                      z2s.at[g, 0], z2r.at[g, 0], z2p)
            rb = exch(g, (bb2 + (1 - bz2) * e, e), comm_z2.at[g, 1],
                      z2s.at[g, 1], z2r.at[g, 1], z2p)
            rt.start()
            rb.start()
            return rt, rb

        def z2_finish(g, h):
            h[0].wait()
            out_ref[pl.ds(tf, e), pl.ds(g * nh, nh)] += comm_z2[g, 0]
            h[1].wait()
            out_ref[pl.ds(bf, e), pl.ds(g * nh, nh)] += comm_z2[g, 1]

        def silu(g):
            for base in (tf, bf):
                zv = out_ref[pl.ds(base, e), pl.ds(g * nh, nh)].astype(jnp.float32)
                sv = zv / (1.0 + jnp.exp(-zv))
                out_ref[pl.ds(base, e), pl.ds(g * nh, nh)] = sv.astype(jnp.bfloat16)

        off0 = bz1 * q + bz2 * e
        off1 = bz1 * q + (1 - bz2) * e
        offL = (1 - bz1) * q

        def sub_info(kind, t):
            off, sz = ((off0, e), (off1, e), (offL, q))[kind % 3]
            if kind < 3:
                cs = lax.rem(w + 1 - t + PLANE, PLANE)
                cr_ = lax.rem(w - t + PLANE, PLANE)
                extra, peer = 0, p_right
            else:
                cs = lax.rem(w - 1 + t + PLANE, PLANE)
                cr_ = lax.rem(w + t, PLANE)
                extra, peer = half, p_left
            return cs * pch + extra + off, cr_ * pch + extra + off, sz, peer

        def sub_start(g, kind, t):
            sbase, _, sz, peer = sub_info(kind, t)
            rd = exch(g, (sbase, sz), zslc(g, sbase, sz),
                      ags.at[g, kind, t], agr.at[g, kind, t], peer)
            rd.start()
            return rd

        def sub_finish(g, kind, t, rd):
            rd.wait()
            return sub_start(g, kind, t + 1) if t + 1 < PLANE - 1 else None

        def ag_open(g):
            r0 = sub_start(g, 0, 0)
            l0 = sub_start(g, 3, 0)
            t2 = exch(g, (tf, e), zslc(g, tf, e),
                      g2s.at[g, 0], g2r.at[g, 0], z2p)
            b2 = exch(g, (bf, e), zslc(g, bf, e),
                      g2s.at[g, 1], g2r.at[g, 1], z2p)
            t2.start()
            b2.start()
            return [r0, l0, t2, b2]

        def ag_mid(g, st):
            r0, l0, t2, b2 = st
            t2.wait()
            b2.wait()
            t1 = exch(g, (tb2, q), zslc(g, tb2, q),
                      g1s.at[g, 0], g1r.at[g, 0], z1p)
            b1 = exch(g, (bb2, q), zslc(g, bb2, q),
                      g1s.at[g, 1], g1r.at[g, 1], z1p)
            t1.start()
            b1.start()
            r1 = sub_start(g, 1, 0)
            l1 = sub_start(g, 4, 0)
            r0 = sub_finish(g, 0, 0, r0)
            l0 = sub_finish(g, 3, 0, l0)
            r1 = sub_finish(g, 1, 0, r1)
            l1 = sub_finish(g, 4, 0, l1)
            r0 = sub_finish(g, 0, 1, r0)
            l0 = sub_finish(g, 3, 1, l0)
            return [r0, l0, r1, l1, t1, b1]

        def ag_close(g, st):
            r0, l0, r1, l1, t1, b1 = st
            t1.wait()
            b1.wait()
            rL = sub_start(g, 2, 0)
            lL = sub_start(g, 5, 0)
            r1 = sub_finish(g, 1, 1, r1)
            l1 = sub_finish(g, 4, 1, l1)
            sub_finish(g, 0, 2, r0)
            sub_finish(g, 3, 2, l0)
            rL = sub_finish(g, 2, 0, rL)
            lL = sub_finish(g, 5, 0, lL)
            sub_finish(g, 1, 2, r1)
            sub_finish(g, 4, 2, l1)
            rL = sub_finish(g, 2, 1, rL)
            lL = sub_finish(g, 5, 1, lL)
            sub_finish(g, 2, 2, rL)
            sub_finish(g, 5, 2, lL)

        mm_slab(w)
        pl.semaphore_wait(barrier, 4)
        ha = rs_starts(0)
        mm_slab(lax.rem(w + PLANE - 1, PLANE))
        mm_slab(lax.rem(w + 1, PLANE))
        mm_slab(lax.rem(w + 2, PLANE))
        ha = rs_round(0, 0, ha)
        hb = rs_starts(1)
        ha = rs_round(0, 1, ha)
        hb = rs_round(1, 0, hb)
        rs_finish(0, 0, 2, ha[0])
        rs_finish(0, 2, 2, ha[1])
        az1 = z1_start(0)
        rs_finish(0, 1, 2, ha[2])
        rs_finish(0, 3, 2, ha[3])
        hb = rs_round(1, 1, hb)
        z1_finish(0, az1)
        az2 = z2_start(0)
        rs_finish(1, 0, 2, hb[0])
        rs_finish(1, 2, 2, hb[1])
        bz1h = z1_start(1)
        rs_finish(1, 1, 2, hb[2])
        rs_finish(1, 3, 2, hb[3])
        z2_finish(0, az2)
        silu(0)
        ag_a = ag_open(0)
        z1_finish(1, bz1h)
        bz2h = z2_start(1)
        ag_a = ag_mid(0, ag_a)
        z2_finish(1, bz2h)
        silu(1)
        ag_b = ag_open(1)
        ag_close(0, ag_a)
        ag_b = ag_mid(1, ag_b)
        ag_close(1, ag_b)

    n_steps = PLANE - 1
    return pl.pallas_call(
        body,
        out_shape=jax.ShapeDtypeStruct((M, N), jnp.bfloat16),
        in_specs=[
            pl.BlockSpec(memory_space=pltpu.VMEM),
            pl.BlockSpec(memory_space=pltpu.VMEM),
        ],
        out_specs=pl.BlockSpec(memory_space=pltpu.VMEM),
        scratch_shapes=[
            pltpu.VMEM((K, N), jnp.bfloat16),
            pltpu.VMEM((2, 4, n_steps, q, nh), jnp.bfloat16),
            pltpu.VMEM((2, 2, q, nh), jnp.bfloat16),
            pltpu.VMEM((2, 2, e, nh), jnp.bfloat16),
            pltpu.SemaphoreType.DMA((2, 4, n_steps)),
            pltpu.SemaphoreType.DMA((2, 4, n_steps)),
            pltpu.SemaphoreType.DMA((2, 2)),
            pltpu.SemaphoreType.DMA((2, 2)),
            pltpu.SemaphoreType.DMA((2, 2)),
            pltpu.SemaphoreType.DMA((2, 2)),
            pltpu.SemaphoreType.DMA((2, 2)),
            pltpu.SemaphoreType.DMA((2, 2)),
            pltpu.SemaphoreType.DMA((2, 2)),
            pltpu.SemaphoreType.DMA((2, 2)),
            pltpu.SemaphoreType.DMA((2, 6, n_steps)),
            pltpu.SemaphoreType.DMA((2, 6, n_steps)),
        ],
        compiler_params=pltpu.CompilerParams(
            collective_id=0,
            vmem_limit_bytes=100 * 1024 * 1024,
        ),
    )(A, B)
